# baseline (speedup 1.0000x reference)
import functools

import jax
import jax.numpy as jnp
from jax import lax
from jax.experimental import pallas as pl
from jax.experimental.pallas import tpu as pltpu
from jax.experimental.pallas import tpu_sc as plsc

_NC = 2
_NS = 16
_NW = _NC * _NS

_BN = 2048


def _mesh():
    return plsc.VectorSubcoreMesh(core_axis_name="c", subcore_axis_name="s")


def _sc_params():
    return pltpu.CompilerParams(needs_layout_passes=False)


def _dense1(x, Ws, bb, A1, ab1, A2, ab2):
    N, DIN = x.shape
    HF = Ws.shape[0]
    grid = (pl.cdiv(N, _BN),)

    def body(x_ref, w_ref, bb_ref, a1_ref, ab1_ref, a2_ref, ab2_ref,
             f_out, a1_out, a2_out):
        f = lax.dot_general(w_ref[...], x_ref[...], (((1,), (1,)), ((), ())),
                            preferred_element_type=jnp.float32)
        f = f + bb_ref[:, 0:1]
        f_out[...] = f
        a1_out[...] = lax.dot_general(a1_ref[...], f, (((1,), (0,)), ((), ())),
                                      preferred_element_type=jnp.float32) + ab1_ref[:, 0:1]
        a2_out[...] = lax.dot_general(a2_ref[...], f, (((1,), (0,)), ((), ())),
                                      preferred_element_type=jnp.float32) + ab2_ref[:, 0:1]

    return pl.pallas_call(
        body,
        grid=grid,
        in_specs=[
            pl.BlockSpec((_BN, DIN), lambda i: (i, 0)),
            pl.BlockSpec((HF, DIN), lambda i: (0, 0)),
            pl.BlockSpec((HF, 128), lambda i: (0, 0)),
            pl.BlockSpec((8, HF), lambda i: (0, 0)),
            pl.BlockSpec((8, 128), lambda i: (0, 0)),
            pl.BlockSpec((8, HF), lambda i: (0, 0)),
            pl.BlockSpec((8, 128), lambda i: (0, 0)),
        ],
        out_specs=[
            pl.BlockSpec((HF, _BN), lambda i: (0, i)),
            pl.BlockSpec((8, _BN), lambda i: (0, i)),
            pl.BlockSpec((8, _BN), lambda i: (0, i)),
        ],
        out_shape=[
            jax.ShapeDtypeStruct((HF, N), jnp.float32),
            jax.ShapeDtypeStruct((8, N), jnp.float32),
            jax.ShapeDtypeStruct((8, N), jnp.float32),
        ],
    )(x, Ws, bb, A1, ab1, A2, ab2)


def _dense2(xT, Ws, bb, A1, ab1, A2, ab2):
    F, N = xT.shape
    HF = Ws.shape[0]
    grid = (pl.cdiv(N, _BN),)

    def body(x_ref, w_ref, bb_ref, a1_ref, ab1_ref, a2_ref, ab2_ref,
             f_out, a1_out, a2_out):
        x2 = jnp.maximum(x_ref[...], 0.0)
        f = lax.dot_general(w_ref[...], x2, (((1,), (0,)), ((), ())),
                            preferred_element_type=jnp.float32)
        f = f + bb_ref[:, 0:1]
        f_out[...] = f
        a1_out[...] = lax.dot_general(a1_ref[...], f, (((1,), (0,)), ((), ())),
                                      preferred_element_type=jnp.float32) + ab1_ref[:, 0:1]
        a2_out[...] = lax.dot_general(a2_ref[...], f, (((1,), (0,)), ((), ())),
                                      preferred_element_type=jnp.float32) + ab2_ref[:, 0:1]

    return pl.pallas_call(
        body,
        grid=grid,
        in_specs=[
            pl.BlockSpec((F, _BN), lambda i: (0, i)),
            pl.BlockSpec((HF, F), lambda i: (0, 0)),
            pl.BlockSpec((HF, 128), lambda i: (0, 0)),
            pl.BlockSpec((8, HF), lambda i: (0, 0)),
            pl.BlockSpec((8, 128), lambda i: (0, 0)),
            pl.BlockSpec((8, HF), lambda i: (0, 0)),
            pl.BlockSpec((8, 128), lambda i: (0, 0)),
        ],
        out_specs=[
            pl.BlockSpec((HF, _BN), lambda i: (0, i)),
            pl.BlockSpec((8, _BN), lambda i: (0, i)),
            pl.BlockSpec((8, _BN), lambda i: (0, i)),
        ],
        out_shape=[
            jax.ShapeDtypeStruct((HF, N), jnp.float32),
            jax.ShapeDtypeStruct((8, N), jnp.float32),
            jax.ShapeDtypeStruct((8, N), jnp.float32),
        ],
    )(xT, Ws, bb, A1, ab1, A2, ab2)


def _sumrecip(spart):
    R, N = spart.shape
    half = R // 2
    grid = (pl.cdiv(N, _BN),)

    def body(s_ref, r_out):
        s = s_ref[...]
        s0 = jnp.sum(s[:half], axis=0, keepdims=True)
        s1 = jnp.sum(s[half:], axis=0, keepdims=True)
        r_out[0:1, :] = 1.0 / s0
        r_out[1:2, :] = 1.0 / s1
        r_out[2:8, :] = jnp.zeros_like(r_out[2:8, :])

    return pl.pallas_call(
        body,
        grid=grid,
        in_specs=[pl.BlockSpec((R, _BN), lambda i: (0, i))],
        out_specs=pl.BlockSpec((8, _BN), lambda i: (0, i)),
        out_shape=jax.ShapeDtypeStruct((8, N), jnp.float32),
    )(spart)


def _finish(pT):
    F, N = pT.shape
    BNf = 512
    grid = (pl.cdiv(N, BNf),)

    def body(p_ref, o_ref):
        y = jnp.maximum(p_ref[...], 0.0)
        o_ref[...] = y.T

    return pl.pallas_call(
        body,
        grid=grid,
        in_specs=[pl.BlockSpec((F, BNf), lambda i: (0, i))],
        out_specs=pl.BlockSpec((BNf, F), lambda i: (i, 0)),
        out_shape=jax.ShapeDtypeStruct((N, F), jnp.float32),
    )(pT)


def _make_att(N, E, from_packed):
    CH = E // _NW
    twoN = 2 * N

    out_type = [
        jax.ShapeDtypeStruct((2 * E,), jnp.float32),
        jax.ShapeDtypeStruct((2 * _NW * N,), jnp.float32),
    ]
    scratch = [
        pltpu.VMEM((twoN,), jnp.float32),
        pltpu.VMEM((twoN,), jnp.float32),
        pltpu.VMEM((twoN,), jnp.float32),
        pltpu.VMEM((2 * CH,), jnp.float32),
        pltpu.VMEM((CH,), jnp.int32),
    ]
    if not from_packed:
        out_type.append(jax.ShapeDtypeStruct((E,), jnp.int32))
        scratch.append(pltpu.VMEM((CH,), jnp.int32))
        scratch.append(pltpu.VMEM((CH,), jnp.int32))

    def body(av1_ref, av2_ref, edges_ref, *refs):
        if from_packed:
            e_out, s_out, a1t, a2t, s_loc, e_v, pck_v = refs
        else:
            (e_out, s_out, pck_out,
             a1t, a2t, s_loc, e_v, pck_v, src_v, dst_v) = refs
        wid = lax.axis_index("s") * _NC + lax.axis_index("c")
        base = wid * CH

        pltpu.sync_copy(av1_ref.at[pl.ds(0, twoN)], a1t)
        pltpu.sync_copy(av2_ref.at[pl.ds(0, twoN)], a2t)
        if from_packed:
            pltpu.sync_copy(edges_ref.at[pl.ds(base, CH)], pck_v)
        else:
            pltpu.sync_copy(edges_ref.at[pl.ds(base, CH)], src_v)
            pltpu.sync_copy(edges_ref.at[pl.ds(E + base, CH)], dst_v)

        zero16 = jnp.zeros((16,), jnp.float32)

        @plsc.parallel_loop(0, twoN, 16, unroll=8)
        def _zero(off):
            s_loc[pl.ds(off, 16)] = zero16

        @plsc.parallel_loop(0, CH, 16, unroll=8)
        def _edges(off):
            if from_packed:
                pk = pck_v[pl.ds(off, 16)]
                s16 = pk >> 14
                d16 = pk & 16383
            else:
                s16 = src_v[pl.ds(off, 16)]
                d16 = dst_v[pl.ds(off, 16)]
                pck_v[pl.ds(off, 16)] = (s16 << 14) | d16
            for h in range(2):
                a1v = plsc.load_gather(a1t, [s16 + (h * N)])
                a2v = plsc.load_gather(a2t, [d16 + (h * N)])
                v = a1v + a2v
                v = jnp.where(v > 0.0, v, 0.01 * v)
                ev = jnp.exp(v)
                e_v[pl.ds(h * CH + off, 16)] = ev
                plsc.addupdate_scatter(s_loc, [s16 + (h * N)], ev)

        pltpu.sync_copy(e_v.at[pl.ds(0, CH)], e_out.at[pl.ds(base, CH)])
        pltpu.sync_copy(e_v.at[pl.ds(CH, CH)], e_out.at[pl.ds(E + base, CH)])
        pltpu.sync_copy(s_loc.at[pl.ds(0, N)],
                        s_out.at[pl.ds(wid * N, N)])
        pltpu.sync_copy(s_loc.at[pl.ds(N, N)],
                        s_out.at[pl.ds(_NW * N + wid * N, N)])
        if not from_packed:
            pltpu.sync_copy(pck_v, pck_out.at[pl.ds(base, CH)])

    return pl.kernel(body, out_type=tuple(out_type), mesh=_mesh(),
                     compiler_params=_sc_params(), scratch_types=scratch)


def _make_agg(N, E, F_all, CH2):
    K = F_all // _NW
    half = F_all // 2

    NCH = E // CH2

    scratch = [
        pltpu.VMEM((K * N,), jnp.float32),
        pltpu.VMEM((N,), jnp.float32),
        pltpu.VMEM((K * N,), jnp.float32),
        pltpu.VMEM((CH2,), jnp.int32),
        pltpu.VMEM((CH2,), jnp.int32),
        pltpu.VMEM((CH2,), jnp.float32),
        pltpu.VMEM((CH2,), jnp.float32),
        pltpu.SemaphoreType.DMA,
        pltpu.SemaphoreType.DMA,
    ]

    def body(fT_ref, e_ref, pck_ref, r_ref, out_ref,
             f_t, r_t, out_t, pck_b0, pck_b1, e_b0, e_b1, sem_p, sem_e):
        wid = lax.axis_index("s") * _NC + lax.axis_index("c")
        c0 = wid * K
        h = c0 // half
        pck_bufs = (pck_b0, pck_b1)
        e_bufs = (e_b0, e_b1)

        pltpu.sync_copy(fT_ref.at[pl.ds(c0 * N, K * N)], f_t)
        pltpu.sync_copy(r_ref.at[pl.ds(h * N, N)], r_t)

        zero16 = jnp.zeros((16,), jnp.float32)

        @plsc.parallel_loop(0, K * N, 16, unroll=8)
        def _zero(off):
            out_t[pl.ds(off, 16)] = zero16

        def start(ci, b):
            pltpu.async_copy(pck_ref.at[pl.ds(ci * CH2, CH2)],
                             pck_bufs[b], sem_p)
            pltpu.async_copy(e_ref.at[pl.ds(h * E + ci * CH2, CH2)],
                             e_bufs[b], sem_e)

        def wait(ci, b):
            pltpu.make_async_copy(pck_ref.at[pl.ds(ci * CH2, CH2)],
                                  pck_bufs[b], sem_p).wait()
            pltpu.make_async_copy(e_ref.at[pl.ds(h * E + ci * CH2, CH2)],
                                  e_bufs[b], sem_e).wait()

        start(0, 0)

        def pair_body(cp, _):
            for b in range(2):
                ci = cp * 2 + b

                @pl.when(ci + 1 < NCH)
                def _():
                    start(ci + 1, 1 - b)

                wait(ci, b)
                pck_b = pck_bufs[b]
                e_b = e_bufs[b]

                @plsc.parallel_loop(0, CH2, 16, unroll=16)
                def _edges(off):
                    pk = pck_b[pl.ds(off, 16)]
                    s16 = pk >> 14
                    d16 = pk & 16383
                    ev = e_b[pl.ds(off, 16)]
                    rv = plsc.load_gather(r_t, [s16])
                    att = ev * rv
                    for c in range(K):
                        fv = plsc.load_gather(f_t, [d16 + (c * N)])
                        plsc.addupdate_scatter(out_t, [s16 + (c * N)],
                                               att * fv)

            return 0

        lax.fori_loop(0, NCH // 2, pair_body, 0)

        pltpu.sync_copy(out_t, out_ref.at[pl.ds(c0 * N, K * N)])

    return pl.kernel(body,
                     out_type=jax.ShapeDtypeStruct((F_all * N,), jnp.float32),
                     mesh=_mesh(), compiler_params=_sc_params(),
                     scratch_types=scratch)


def _blockdiag(aw):
    H, F = aw.shape
    A = jnp.zeros((8, H * F), jnp.float32)
    for h in range(H):
        A = A.at[h, h * F:(h + 1) * F].set(aw[h])
    return A


def _bcast_col(v):
    return jnp.broadcast_to(v.reshape(-1, 1), (v.size, 128)).astype(jnp.float32)


def kernel(features, edge_index, W1, b1, a1w1, a1b1, a2w1, a2b1,
           W2, b2, a1w2, a1b2, a2w2, a2b2):
    N, DIN = features.shape
    E = edge_index.shape[1]
    H, F1, _ = W1.shape
    F2 = W2.shape[1]
    HF1, HF2 = H * F1, H * F2

    Ws1 = W1.reshape(HF1, DIN)
    Ws2 = W2.reshape(HF2, HF1)
    bb1 = _bcast_col(b1)
    bb2 = _bcast_col(b2)
    A1_1, A2_1 = _blockdiag(a1w1), _blockdiag(a2w1)
    A1_2, A2_2 = _blockdiag(a1w2), _blockdiag(a2w2)
    ab1_1 = _bcast_col(jnp.pad(a1b1, (0, 8 - H)))
    ab2_1 = _bcast_col(jnp.pad(a2b1, (0, 8 - H)))
    ab1_2 = _bcast_col(jnp.pad(a1b2, (0, 8 - H)))
    ab2_2 = _bcast_col(jnp.pad(a2b2, (0, 8 - H)))

    att1 = _make_att(N, E, from_packed=False)
    att2 = _make_att(N, E, from_packed=True)
    agg1 = _make_agg(N, E, HF1, 16000)
    agg2 = _make_agg(N, E, HF2, 8000)

    fT1, a1v1, a2v1 = _dense1(features, Ws1, bb1, A1_1, ab1_1, A2_1, ab2_1)
    e1, spart1, pck = att1(a1v1.reshape(-1), a2v1.reshape(-1),
                           edge_index.reshape(-1))
    r1 = _sumrecip(spart1.reshape(2 * _NW, N))
    out1 = agg1(fT1.reshape(-1), e1, pck, r1.reshape(-1))

    fT2, a1v2, a2v2 = _dense2(out1.reshape(HF1, N), Ws2, bb2,
                              A1_2, ab1_2, A2_2, ab2_2)
    e2, spart2 = att2(a1v2.reshape(-1), a2v2.reshape(-1), pck)
    r2 = _sumrecip(spart2.reshape(2 * _NW, N))
    out2 = agg2(fT2.reshape(-1), e2, pck, r2.reshape(-1))

    return _finish(out2.reshape(HF2, N))


# speedup vs baseline: 1.0936x; 1.0936x over previous
import functools

import jax
import jax.numpy as jnp
from jax import lax
from jax.experimental import pallas as pl
from jax.experimental.pallas import tpu as pltpu
from jax.experimental.pallas import tpu_sc as plsc

_NC = 2
_NS = 16
_NW = _NC * _NS

_BN = 2048


def _mesh():
    return plsc.VectorSubcoreMesh(core_axis_name="c", subcore_axis_name="s")


def _sc_params():
    return pltpu.CompilerParams(needs_layout_passes=False)


def _dense1(x, Ws, bb, A1, ab1, A2, ab2):
    N, DIN = x.shape
    HF = Ws.shape[0]
    grid = (pl.cdiv(N, _BN),)

    def body(x_ref, w_ref, bb_ref, a1_ref, ab1_ref, a2_ref, ab2_ref,
             f_out, a1_out, a2_out):
        f = lax.dot_general(w_ref[...], x_ref[...], (((1,), (1,)), ((), ())),
                            preferred_element_type=jnp.float32)
        f = f + bb_ref[:, 0:1]
        f_out[...] = f
        a1_out[...] = lax.dot_general(a1_ref[...], f, (((1,), (0,)), ((), ())),
                                      preferred_element_type=jnp.float32) + ab1_ref[:, 0:1]
        a2_out[...] = lax.dot_general(a2_ref[...], f, (((1,), (0,)), ((), ())),
                                      preferred_element_type=jnp.float32) + ab2_ref[:, 0:1]

    return pl.pallas_call(
        body,
        grid=grid,
        in_specs=[
            pl.BlockSpec((_BN, DIN), lambda i: (i, 0)),
            pl.BlockSpec((HF, DIN), lambda i: (0, 0)),
            pl.BlockSpec((HF, 128), lambda i: (0, 0)),
            pl.BlockSpec((8, HF), lambda i: (0, 0)),
            pl.BlockSpec((8, 128), lambda i: (0, 0)),
            pl.BlockSpec((8, HF), lambda i: (0, 0)),
            pl.BlockSpec((8, 128), lambda i: (0, 0)),
        ],
        out_specs=[
            pl.BlockSpec((HF, _BN), lambda i: (0, i)),
            pl.BlockSpec((8, _BN), lambda i: (0, i)),
            pl.BlockSpec((8, _BN), lambda i: (0, i)),
        ],
        out_shape=[
            jax.ShapeDtypeStruct((HF, N), jnp.float32),
            jax.ShapeDtypeStruct((8, N), jnp.float32),
            jax.ShapeDtypeStruct((8, N), jnp.float32),
        ],
    )(x, Ws, bb, A1, ab1, A2, ab2)


def _dense2(xT, Ws, bb, A1, ab1, A2, ab2):
    F, N = xT.shape
    HF = Ws.shape[0]
    grid = (pl.cdiv(N, _BN),)

    def body(x_ref, w_ref, bb_ref, a1_ref, ab1_ref, a2_ref, ab2_ref,
             f_out, a1_out, a2_out):
        x2 = jnp.maximum(x_ref[...], 0.0)
        f = lax.dot_general(w_ref[...], x2, (((1,), (0,)), ((), ())),
                            preferred_element_type=jnp.float32)
        f = f + bb_ref[:, 0:1]
        f_out[...] = f
        a1_out[...] = lax.dot_general(a1_ref[...], f, (((1,), (0,)), ((), ())),
                                      preferred_element_type=jnp.float32) + ab1_ref[:, 0:1]
        a2_out[...] = lax.dot_general(a2_ref[...], f, (((1,), (0,)), ((), ())),
                                      preferred_element_type=jnp.float32) + ab2_ref[:, 0:1]

    return pl.pallas_call(
        body,
        grid=grid,
        in_specs=[
            pl.BlockSpec((F, _BN), lambda i: (0, i)),
            pl.BlockSpec((HF, F), lambda i: (0, 0)),
            pl.BlockSpec((HF, 128), lambda i: (0, 0)),
            pl.BlockSpec((8, HF), lambda i: (0, 0)),
            pl.BlockSpec((8, 128), lambda i: (0, 0)),
            pl.BlockSpec((8, HF), lambda i: (0, 0)),
            pl.BlockSpec((8, 128), lambda i: (0, 0)),
        ],
        out_specs=[
            pl.BlockSpec((HF, _BN), lambda i: (0, i)),
            pl.BlockSpec((8, _BN), lambda i: (0, i)),
            pl.BlockSpec((8, _BN), lambda i: (0, i)),
        ],
        out_shape=[
            jax.ShapeDtypeStruct((HF, N), jnp.float32),
            jax.ShapeDtypeStruct((8, N), jnp.float32),
            jax.ShapeDtypeStruct((8, N), jnp.float32),
        ],
    )(xT, Ws, bb, A1, ab1, A2, ab2)


def _sumrecip(spart):
    R, N = spart.shape
    half = R // 2
    grid = (pl.cdiv(N, _BN),)

    def body(s_ref, r_out):
        s = s_ref[...]
        s0 = jnp.sum(s[:half], axis=0, keepdims=True)
        s1 = jnp.sum(s[half:], axis=0, keepdims=True)
        r_out[0:1, :] = 1.0 / s0
        r_out[1:2, :] = 1.0 / s1
        r_out[2:8, :] = jnp.zeros_like(r_out[2:8, :])

    return pl.pallas_call(
        body,
        grid=grid,
        in_specs=[pl.BlockSpec((R, _BN), lambda i: (0, i))],
        out_specs=pl.BlockSpec((8, _BN), lambda i: (0, i)),
        out_shape=jax.ShapeDtypeStruct((8, N), jnp.float32),
    )(spart)


def _finish(pT):
    F, N = pT.shape
    BNf = 512
    grid = (pl.cdiv(N, BNf),)

    def body(p_ref, o_ref):
        y = jnp.maximum(p_ref[...], 0.0)
        o_ref[...] = y.T

    return pl.pallas_call(
        body,
        grid=grid,
        in_specs=[pl.BlockSpec((F, BNf), lambda i: (0, i))],
        out_specs=pl.BlockSpec((BNf, F), lambda i: (i, 0)),
        out_shape=jax.ShapeDtypeStruct((N, F), jnp.float32),
    )(pT)


def _make_att(N, E, from_packed):
    CH = E // _NW
    twoN = 2 * N

    out_type = [
        jax.ShapeDtypeStruct((2 * E,), jnp.float32),
        jax.ShapeDtypeStruct((2 * _NW * N,), jnp.float32),
    ]
    scratch = [
        pltpu.VMEM((twoN,), jnp.float32),
        pltpu.VMEM((twoN,), jnp.float32),
        pltpu.VMEM((twoN,), jnp.float32),
        pltpu.VMEM((2 * CH,), jnp.float32),
        pltpu.VMEM((CH,), jnp.int32),
    ]
    if not from_packed:
        out_type.append(jax.ShapeDtypeStruct((E,), jnp.int32))
        scratch.append(pltpu.VMEM((CH,), jnp.int32))
        scratch.append(pltpu.VMEM((CH,), jnp.int32))

    def body(av1_ref, av2_ref, edges_ref, *refs):
        if from_packed:
            e_out, s_out, a1t, a2t, s_loc, e_v, pck_v = refs
        else:
            (e_out, s_out, pck_out,
             a1t, a2t, s_loc, e_v, pck_v, src_v, dst_v) = refs
        wid = lax.axis_index("s") * _NC + lax.axis_index("c")
        base = wid * CH

        pltpu.sync_copy(av1_ref.at[pl.ds(0, twoN)], a1t)
        pltpu.sync_copy(av2_ref.at[pl.ds(0, twoN)], a2t)
        if from_packed:
            pltpu.sync_copy(edges_ref.at[pl.ds(base, CH)], pck_v)
        else:
            pltpu.sync_copy(edges_ref.at[pl.ds(base, CH)], src_v)
            pltpu.sync_copy(edges_ref.at[pl.ds(E + base, CH)], dst_v)

        zero16 = jnp.zeros((16,), jnp.float32)

        @plsc.parallel_loop(0, twoN, 16, unroll=8)
        def _zero(off):
            s_loc[pl.ds(off, 16)] = zero16

        @plsc.parallel_loop(0, CH, 16, unroll=8)
        def _edges(off):
            if from_packed:
                pk = pck_v[pl.ds(off, 16)]
                s16 = pk >> 14
                d16 = pk & 16383
            else:
                s16 = src_v[pl.ds(off, 16)]
                d16 = dst_v[pl.ds(off, 16)]
                pck_v[pl.ds(off, 16)] = (s16 << 14) | d16
            for h in range(2):
                a1v = plsc.load_gather(a1t, [s16 + (h * N)])
                a2v = plsc.load_gather(a2t, [d16 + (h * N)])
                v = a1v + a2v
                v = jnp.where(v > 0.0, v, 0.01 * v)
                ev = jnp.exp(v)
                e_v[pl.ds(h * CH + off, 16)] = ev
                plsc.addupdate_scatter(s_loc, [s16 + (h * N)], ev)

        pltpu.sync_copy(e_v.at[pl.ds(0, CH)], e_out.at[pl.ds(base, CH)])
        pltpu.sync_copy(e_v.at[pl.ds(CH, CH)], e_out.at[pl.ds(E + base, CH)])
        pltpu.sync_copy(s_loc.at[pl.ds(0, N)],
                        s_out.at[pl.ds(wid * N, N)])
        pltpu.sync_copy(s_loc.at[pl.ds(N, N)],
                        s_out.at[pl.ds(_NW * N + wid * N, N)])
        if not from_packed:
            pltpu.sync_copy(pck_v, pck_out.at[pl.ds(base, CH)])

    return pl.kernel(body, out_type=tuple(out_type), mesh=_mesh(),
                     compiler_params=_sc_params(), scratch_types=scratch)


def _make_agg(N, E, F_all, CH2):
    K = F_all // _NW
    half = F_all // 2

    NCH = E // CH2

    scratch = [
        pltpu.VMEM((K * N,), jnp.float32),
        pltpu.VMEM((N,), jnp.float32),
        pltpu.VMEM((K * N,), jnp.float32),
        pltpu.VMEM((CH2,), jnp.int32),
        pltpu.VMEM((CH2,), jnp.int32),
        pltpu.VMEM((CH2,), jnp.float32),
        pltpu.VMEM((CH2,), jnp.float32),
        pltpu.SemaphoreType.DMA,
        pltpu.SemaphoreType.DMA,
    ]

    def body(fT_ref, e_ref, pck_ref, r_ref, out_ref,
             f_t, r_t, out_t, pck_b0, pck_b1, e_b0, e_b1, sem_p, sem_e):
        wid = lax.axis_index("s") * _NC + lax.axis_index("c")
        c0 = wid * K
        h = c0 // half
        pck_bufs = (pck_b0, pck_b1)
        e_bufs = (e_b0, e_b1)

        pltpu.sync_copy(fT_ref.at[pl.ds(c0 * N, K * N)], f_t)
        pltpu.sync_copy(r_ref.at[pl.ds(h * N, N)], r_t)

        zero16 = jnp.zeros((16,), jnp.float32)

        @plsc.parallel_loop(0, K * N, 16, unroll=8)
        def _zero(off):
            out_t[pl.ds(off, 16)] = zero16

        def start(ci, b):
            pltpu.async_copy(pck_ref.at[pl.ds(ci * CH2, CH2)],
                             pck_bufs[b], sem_p)
            pltpu.async_copy(e_ref.at[pl.ds(h * E + ci * CH2, CH2)],
                             e_bufs[b], sem_e)

        def wait(ci, b):
            pltpu.make_async_copy(pck_ref.at[pl.ds(ci * CH2, CH2)],
                                  pck_bufs[b], sem_p).wait()
            pltpu.make_async_copy(e_ref.at[pl.ds(h * E + ci * CH2, CH2)],
                                  e_bufs[b], sem_e).wait()

        start(0, 0)

        def pair_body(cp, _):
            for b in range(2):
                ci = cp * 2 + b

                @pl.when(ci + 1 < NCH)
                def _():
                    start(ci + 1, 1 - b)

                wait(ci, b)
                pck_b = pck_bufs[b]
                e_b = e_bufs[b]

                @plsc.parallel_loop(0, CH2, 16, unroll=8)
                def _edges(off):
                    pk = pck_b[pl.ds(off, 16)]
                    s16 = pk >> 14
                    d16 = pk & 16383
                    ev = e_b[pl.ds(off, 16)]
                    rv = plsc.load_gather(r_t, [s16])
                    att = ev * rv
                    for c in range(K):
                        fv = plsc.load_gather(f_t, [d16 + (c * N)])
                        plsc.addupdate_scatter(out_t, [s16 + (c * N)],
                                               att * fv)

            return 0

        lax.fori_loop(0, NCH // 2, pair_body, 0)

        pltpu.sync_copy(out_t, out_ref.at[pl.ds(c0 * N, K * N)])

    return pl.kernel(body,
                     out_type=jax.ShapeDtypeStruct((F_all * N,), jnp.float32),
                     mesh=_mesh(), compiler_params=_sc_params(),
                     scratch_types=scratch)


def _blockdiag(aw):
    H, F = aw.shape
    A = jnp.zeros((8, H * F), jnp.float32)
    for h in range(H):
        A = A.at[h, h * F:(h + 1) * F].set(aw[h])
    return A


def _bcast_col(v):
    return jnp.broadcast_to(v.reshape(-1, 1), (v.size, 128)).astype(jnp.float32)


def kernel(features, edge_index, W1, b1, a1w1, a1b1, a2w1, a2b1,
           W2, b2, a1w2, a1b2, a2w2, a2b2):
    N, DIN = features.shape
    E = edge_index.shape[1]
    H, F1, _ = W1.shape
    F2 = W2.shape[1]
    HF1, HF2 = H * F1, H * F2

    Ws1 = W1.reshape(HF1, DIN)
    Ws2 = W2.reshape(HF2, HF1)
    bb1 = _bcast_col(b1)
    bb2 = _bcast_col(b2)
    A1_1, A2_1 = _blockdiag(a1w1), _blockdiag(a2w1)
    A1_2, A2_2 = _blockdiag(a1w2), _blockdiag(a2w2)
    ab1_1 = _bcast_col(jnp.pad(a1b1, (0, 8 - H)))
    ab2_1 = _bcast_col(jnp.pad(a2b1, (0, 8 - H)))
    ab1_2 = _bcast_col(jnp.pad(a1b2, (0, 8 - H)))
    ab2_2 = _bcast_col(jnp.pad(a2b2, (0, 8 - H)))

    att1 = _make_att(N, E, from_packed=False)
    att2 = _make_att(N, E, from_packed=True)
    agg1 = _make_agg(N, E, HF1, 16000)
    agg2 = _make_agg(N, E, HF2, 8000)

    fT1, a1v1, a2v1 = _dense1(features, Ws1, bb1, A1_1, ab1_1, A2_1, ab2_1)
    e1, spart1, pck = att1(a1v1.reshape(-1), a2v1.reshape(-1),
                           edge_index.reshape(-1))
    r1 = _sumrecip(spart1.reshape(2 * _NW, N))
    out1 = agg1(fT1.reshape(-1), e1, pck, r1.reshape(-1))

    fT2, a1v2, a2v2 = _dense2(out1.reshape(HF1, N), Ws2, bb2,
                              A1_2, ab1_2, A2_2, ab2_2)
    e2, spart2 = att2(a1v2.reshape(-1), a2v2.reshape(-1), pck)
    r2 = _sumrecip(spart2.reshape(2 * _NW, N))
    out2 = agg2(fT2.reshape(-1), e2, pck, r2.reshape(-1))

    return _finish(out2.reshape(HF2, N))


# speedup vs baseline: 1.2650x; 1.1567x over previous
import functools

import jax
import jax.numpy as jnp
from jax import lax
from jax.experimental import pallas as pl
from jax.experimental.pallas import tpu as pltpu
from jax.experimental.pallas import tpu_sc as plsc

_NC = 2
_NS = 16
_NW = _NC * _NS

_BN = 2048


def _mesh():
    return plsc.VectorSubcoreMesh(core_axis_name="c", subcore_axis_name="s")


def _sc_params():
    return pltpu.CompilerParams(needs_layout_passes=False)


def _pack_pair(fe, fo):
    ue = lax.bitcast_convert_type(fe.astype(jnp.bfloat16), jnp.uint16)
    uo = lax.bitcast_convert_type(fo.astype(jnp.bfloat16), jnp.uint16)
    w = (uo.astype(jnp.uint32) << 16) | ue.astype(jnp.uint32)
    return lax.bitcast_convert_type(w, jnp.int32)


def _dense1(x, We, Wo, be, bo, A1e, A1o, ab1, A2e, A2o, ab2):
    N, DIN = x.shape
    HFH = We.shape[0]
    grid = (pl.cdiv(N, _BN),)

    def body(x_ref, we_ref, wo_ref, be_ref, bo_ref,
             a1e_ref, a1o_ref, ab1_ref, a2e_ref, a2o_ref, ab2_ref,
             f_out, a1_out, a2_out):
        x = x_ref[...]
        fe = lax.dot_general(we_ref[...], x, (((1,), (1,)), ((), ())),
                             preferred_element_type=jnp.float32) + be_ref[:, 0:1]
        fo = lax.dot_general(wo_ref[...], x, (((1,), (1,)), ((), ())),
                             preferred_element_type=jnp.float32) + bo_ref[:, 0:1]
        f_out[...] = _pack_pair(fe, fo)
        a1_out[...] = (
            lax.dot_general(a1e_ref[...], fe, (((1,), (0,)), ((), ())),
                            preferred_element_type=jnp.float32)
            + lax.dot_general(a1o_ref[...], fo, (((1,), (0,)), ((), ())),
                              preferred_element_type=jnp.float32)
            + ab1_ref[:, 0:1])
        a2_out[...] = (
            lax.dot_general(a2e_ref[...], fe, (((1,), (0,)), ((), ())),
                            preferred_element_type=jnp.float32)
            + lax.dot_general(a2o_ref[...], fo, (((1,), (0,)), ((), ())),
                              preferred_element_type=jnp.float32)
            + ab2_ref[:, 0:1])

    return pl.pallas_call(
        body,
        grid=grid,
        in_specs=[
            pl.BlockSpec((_BN, DIN), lambda i: (i, 0)),
            pl.BlockSpec((HFH, DIN), lambda i: (0, 0)),
            pl.BlockSpec((HFH, DIN), lambda i: (0, 0)),
            pl.BlockSpec((HFH, 128), lambda i: (0, 0)),
            pl.BlockSpec((HFH, 128), lambda i: (0, 0)),
            pl.BlockSpec((8, HFH), lambda i: (0, 0)),
            pl.BlockSpec((8, HFH), lambda i: (0, 0)),
            pl.BlockSpec((8, 128), lambda i: (0, 0)),
            pl.BlockSpec((8, HFH), lambda i: (0, 0)),
            pl.BlockSpec((8, HFH), lambda i: (0, 0)),
            pl.BlockSpec((8, 128), lambda i: (0, 0)),
        ],
        out_specs=[
            pl.BlockSpec((HFH, _BN), lambda i: (0, i)),
            pl.BlockSpec((8, _BN), lambda i: (0, i)),
            pl.BlockSpec((8, _BN), lambda i: (0, i)),
        ],
        out_shape=[
            jax.ShapeDtypeStruct((HFH, N), jnp.int32),
            jax.ShapeDtypeStruct((8, N), jnp.float32),
            jax.ShapeDtypeStruct((8, N), jnp.float32),
        ],
    )(x, We, Wo, be, bo, A1e, A1o, ab1, A2e, A2o, ab2)


def _dense2(xT, We, Wo, be, bo, A1e, A1o, ab1, A2e, A2o, ab2):
    F, N = xT.shape
    HFH = We.shape[0]
    grid = (pl.cdiv(N, _BN),)

    def body(x_ref, we_ref, wo_ref, be_ref, bo_ref,
             a1e_ref, a1o_ref, ab1_ref, a2e_ref, a2o_ref, ab2_ref,
             f_out, a1_out, a2_out):
        x2 = jnp.maximum(x_ref[...], 0.0)
        fe = lax.dot_general(we_ref[...], x2, (((1,), (0,)), ((), ())),
                             preferred_element_type=jnp.float32) + be_ref[:, 0:1]
        fo = lax.dot_general(wo_ref[...], x2, (((1,), (0,)), ((), ())),
                             preferred_element_type=jnp.float32) + bo_ref[:, 0:1]
        f_out[...] = _pack_pair(fe, fo)
        a1_out[...] = (
            lax.dot_general(a1e_ref[...], fe, (((1,), (0,)), ((), ())),
                            preferred_element_type=jnp.float32)
            + lax.dot_general(a1o_ref[...], fo, (((1,), (0,)), ((), ())),
                              preferred_element_type=jnp.float32)
            + ab1_ref[:, 0:1])
        a2_out[...] = (
            lax.dot_general(a2e_ref[...], fe, (((1,), (0,)), ((), ())),
                            preferred_element_type=jnp.float32)
            + lax.dot_general(a2o_ref[...], fo, (((1,), (0,)), ((), ())),
                              preferred_element_type=jnp.float32)
            + ab2_ref[:, 0:1])

    return pl.pallas_call(
        body,
        grid=grid,
        in_specs=[
            pl.BlockSpec((F, _BN), lambda i: (0, i)),
            pl.BlockSpec((HFH, F), lambda i: (0, 0)),
            pl.BlockSpec((HFH, F), lambda i: (0, 0)),
            pl.BlockSpec((HFH, 128), lambda i: (0, 0)),
            pl.BlockSpec((HFH, 128), lambda i: (0, 0)),
            pl.BlockSpec((8, HFH), lambda i: (0, 0)),
            pl.BlockSpec((8, HFH), lambda i: (0, 0)),
            pl.BlockSpec((8, 128), lambda i: (0, 0)),
            pl.BlockSpec((8, HFH), lambda i: (0, 0)),
            pl.BlockSpec((8, HFH), lambda i: (0, 0)),
            pl.BlockSpec((8, 128), lambda i: (0, 0)),
        ],
        out_specs=[
            pl.BlockSpec((HFH, _BN), lambda i: (0, i)),
            pl.BlockSpec((8, _BN), lambda i: (0, i)),
            pl.BlockSpec((8, _BN), lambda i: (0, i)),
        ],
        out_shape=[
            jax.ShapeDtypeStruct((HFH, N), jnp.int32),
            jax.ShapeDtypeStruct((8, N), jnp.float32),
            jax.ShapeDtypeStruct((8, N), jnp.float32),
        ],
    )(xT, We, Wo, be, bo, A1e, A1o, ab1, A2e, A2o, ab2)


def _sumrecip(spart):
    R, N = spart.shape
    half = R // 2
    grid = (pl.cdiv(N, _BN),)

    def body(s_ref, r_out):
        s = s_ref[...]
        s0 = jnp.sum(s[:half], axis=0, keepdims=True)
        s1 = jnp.sum(s[half:], axis=0, keepdims=True)
        r_out[0:1, :] = 1.0 / s0
        r_out[1:2, :] = 1.0 / s1
        r_out[2:8, :] = jnp.zeros_like(r_out[2:8, :])

    return pl.pallas_call(
        body,
        grid=grid,
        in_specs=[pl.BlockSpec((R, _BN), lambda i: (0, i))],
        out_specs=pl.BlockSpec((8, _BN), lambda i: (0, i)),
        out_shape=jax.ShapeDtypeStruct((8, N), jnp.float32),
    )(spart)


def _finish(pT):
    F, N = pT.shape
    BNf = 512
    grid = (pl.cdiv(N, BNf),)

    def body(p_ref, o_ref):
        y = jnp.maximum(p_ref[...], 0.0)
        o_ref[...] = y.T

    return pl.pallas_call(
        body,
        grid=grid,
        in_specs=[pl.BlockSpec((F, BNf), lambda i: (0, i))],
        out_specs=pl.BlockSpec((BNf, F), lambda i: (i, 0)),
        out_shape=jax.ShapeDtypeStruct((N, F), jnp.float32),
    )(pT)


def _make_att(N, E, from_packed):
    CH = E // _NW
    twoN = 2 * N

    out_type = [
        jax.ShapeDtypeStruct((2 * E,), jnp.float32),
        jax.ShapeDtypeStruct((2 * _NW * N,), jnp.float32),
    ]
    scratch = [
        pltpu.VMEM((twoN,), jnp.float32),
        pltpu.VMEM((twoN,), jnp.float32),
        pltpu.VMEM((twoN,), jnp.float32),
        pltpu.VMEM((2 * CH,), jnp.float32),
        pltpu.VMEM((CH,), jnp.int32),
    ]
    if not from_packed:
        out_type.append(jax.ShapeDtypeStruct((E,), jnp.int32))
        scratch.append(pltpu.VMEM((CH,), jnp.int32))
        scratch.append(pltpu.VMEM((CH,), jnp.int32))

    def body(av1_ref, av2_ref, edges_ref, *refs):
        if from_packed:
            e_out, s_out, a1t, a2t, s_loc, e_v, pck_v = refs
        else:
            (e_out, s_out, pck_out,
             a1t, a2t, s_loc, e_v, pck_v, src_v, dst_v) = refs
        wid = lax.axis_index("s") * _NC + lax.axis_index("c")
        base = wid * CH

        pltpu.sync_copy(av1_ref.at[pl.ds(0, twoN)], a1t)
        pltpu.sync_copy(av2_ref.at[pl.ds(0, twoN)], a2t)
        if from_packed:
            pltpu.sync_copy(edges_ref.at[pl.ds(base, CH)], pck_v)
        else:
            pltpu.sync_copy(edges_ref.at[pl.ds(base, CH)], src_v)
            pltpu.sync_copy(edges_ref.at[pl.ds(E + base, CH)], dst_v)

        zero16 = jnp.zeros((16,), jnp.float32)

        @plsc.parallel_loop(0, twoN, 16, unroll=8)
        def _zero(off):
            s_loc[pl.ds(off, 16)] = zero16

        @plsc.parallel_loop(0, CH, 16, unroll=8)
        def _edges(off):
            if from_packed:
                pk = pck_v[pl.ds(off, 16)]
                s16 = pk >> 14
                d16 = pk & 16383
            else:
                s16 = src_v[pl.ds(off, 16)]
                d16 = dst_v[pl.ds(off, 16)]
                pck_v[pl.ds(off, 16)] = (s16 << 14) | d16
            for h in range(2):
                a1v = plsc.load_gather(a1t, [s16 + (h * N)])
                a2v = plsc.load_gather(a2t, [d16 + (h * N)])
                v = a1v + a2v
                v = jnp.where(v > 0.0, v, 0.01 * v)
                ev = jnp.exp(v)
                e_v[pl.ds(h * CH + off, 16)] = ev
                plsc.addupdate_scatter(s_loc, [s16 + (h * N)], ev)

        pltpu.sync_copy(e_v.at[pl.ds(0, CH)], e_out.at[pl.ds(base, CH)])
        pltpu.sync_copy(e_v.at[pl.ds(CH, CH)], e_out.at[pl.ds(E + base, CH)])
        pltpu.sync_copy(s_loc.at[pl.ds(0, N)],
                        s_out.at[pl.ds(wid * N, N)])
        pltpu.sync_copy(s_loc.at[pl.ds(N, N)],
                        s_out.at[pl.ds(_NW * N + wid * N, N)])
        if not from_packed:
            pltpu.sync_copy(pck_v, pck_out.at[pl.ds(base, CH)])

    return pl.kernel(body, out_type=tuple(out_type), mesh=_mesh(),
                     compiler_params=_sc_params(), scratch_types=scratch)


def _make_agg(N, E, F_all, CH2):
    K = F_all // _NW
    KH = K // 2
    half = F_all // 2

    NCH = E // CH2

    scratch = [
        pltpu.VMEM((KH * N,), jnp.int32),
        pltpu.VMEM((N,), jnp.float32),
        pltpu.VMEM((K * N,), jnp.float32),
        pltpu.VMEM((CH2,), jnp.int32),
        pltpu.VMEM((CH2,), jnp.int32),
        pltpu.VMEM((CH2,), jnp.float32),
        pltpu.VMEM((CH2,), jnp.float32),
        pltpu.SemaphoreType.DMA,
        pltpu.SemaphoreType.DMA,
    ]

    def body(fT_ref, e_ref, pck_ref, r_ref, out_ref,
             f_t, r_t, out_t, pck_b0, pck_b1, e_b0, e_b1, sem_p, sem_e):
        wid = lax.axis_index("s") * _NC + lax.axis_index("c")
        c0 = wid * K
        h = c0 // half
        pck_bufs = (pck_b0, pck_b1)
        e_bufs = (e_b0, e_b1)

        pltpu.sync_copy(fT_ref.at[pl.ds(wid * (KH * N), KH * N)], f_t)
        pltpu.sync_copy(r_ref.at[pl.ds(h * N, N)], r_t)

        zero16 = jnp.zeros((16,), jnp.float32)

        @plsc.parallel_loop(0, K * N, 16, unroll=8)
        def _zero(off):
            out_t[pl.ds(off, 16)] = zero16

        def start(ci, b):
            pltpu.async_copy(pck_ref.at[pl.ds(ci * CH2, CH2)],
                             pck_bufs[b], sem_p)
            pltpu.async_copy(e_ref.at[pl.ds(h * E + ci * CH2, CH2)],
                             e_bufs[b], sem_e)

        def wait(ci, b):
            pltpu.make_async_copy(pck_ref.at[pl.ds(ci * CH2, CH2)],
                                  pck_bufs[b], sem_p).wait()
            pltpu.make_async_copy(e_ref.at[pl.ds(h * E + ci * CH2, CH2)],
                                  e_bufs[b], sem_e).wait()

        start(0, 0)

        def pair_body(cp, _):
            for b in range(2):
                ci = cp * 2 + b

                @pl.when(ci + 1 < NCH)
                def _():
                    start(ci + 1, 1 - b)

                wait(ci, b)
                pck_b = pck_bufs[b]
                e_b = e_bufs[b]

                @plsc.parallel_loop(0, CH2, 16, unroll=8)
                def _edges(off):
                    pk = pck_b[pl.ds(off, 16)]
                    s16 = pk >> 14
                    d16 = pk & 16383
                    ev = e_b[pl.ds(off, 16)]
                    rv = plsc.load_gather(r_t, [s16])
                    att = ev * rv
                    for cp in range(KH):
                        w16 = plsc.load_gather(f_t, [d16 + (cp * N)])
                        flo, fhi = plsc.unpack(
                            plsc.bitcast(w16, jnp.bfloat16),
                            format=plsc.PackFormat.INTERLEAVED)
                        plsc.addupdate_scatter(
                            out_t, [s16 + ((2 * cp) * N)], att * flo)
                        plsc.addupdate_scatter(
                            out_t, [s16 + ((2 * cp + 1) * N)], att * fhi)

            return 0

        lax.fori_loop(0, NCH // 2, pair_body, 0)

        pltpu.sync_copy(out_t, out_ref.at[pl.ds(c0 * N, K * N)])

    return pl.kernel(body,
                     out_type=jax.ShapeDtypeStruct((F_all * N,), jnp.float32),
                     mesh=_mesh(), compiler_params=_sc_params(),
                     scratch_types=scratch)


def _blockdiag(aw):
    H, F = aw.shape
    A = jnp.zeros((8, H * F), jnp.float32)
    for h in range(H):
        A = A.at[h, h * F:(h + 1) * F].set(aw[h])
    return A


def _bcast_col(v):
    return jnp.broadcast_to(v.reshape(-1, 1), (v.size, 128)).astype(jnp.float32)


def kernel(features, edge_index, W1, b1, a1w1, a1b1, a2w1, a2b1,
           W2, b2, a1w2, a1b2, a2w2, a2b2):
    N, DIN = features.shape
    E = edge_index.shape[1]
    H, F1, _ = W1.shape
    F2 = W2.shape[1]
    HF1, HF2 = H * F1, H * F2

    Ws1 = W1.reshape(HF1, DIN)
    Ws2 = W2.reshape(HF2, HF1)
    bs1 = b1.reshape(HF1)
    bs2 = b2.reshape(HF2)
    A1_1, A2_1 = _blockdiag(a1w1), _blockdiag(a2w1)
    A1_2, A2_2 = _blockdiag(a1w2), _blockdiag(a2w2)
    ab1_1 = _bcast_col(jnp.pad(a1b1, (0, 8 - H)))
    ab2_1 = _bcast_col(jnp.pad(a2b1, (0, 8 - H)))
    ab1_2 = _bcast_col(jnp.pad(a1b2, (0, 8 - H)))
    ab2_2 = _bcast_col(jnp.pad(a2b2, (0, 8 - H)))

    att1 = _make_att(N, E, from_packed=False)
    att2 = _make_att(N, E, from_packed=True)
    agg1 = _make_agg(N, E, HF1, 16000)
    agg2 = _make_agg(N, E, HF2, 8000)

    fp1, a1v1, a2v1 = _dense1(
        features, Ws1[0::2], Ws1[1::2],
        _bcast_col(bs1[0::2]), _bcast_col(bs1[1::2]),
        A1_1[:, 0::2], A1_1[:, 1::2], ab1_1,
        A2_1[:, 0::2], A2_1[:, 1::2], ab2_1)
    e1, spart1, pck = att1(a1v1.reshape(-1), a2v1.reshape(-1),
                           edge_index.reshape(-1))
    r1 = _sumrecip(spart1.reshape(2 * _NW, N))
    out1 = agg1(fp1.reshape(-1), e1, pck, r1.reshape(-1))

    fp2, a1v2, a2v2 = _dense2(
        out1.reshape(HF1, N), Ws2[0::2], Ws2[1::2],
        _bcast_col(bs2[0::2]), _bcast_col(bs2[1::2]),
        A1_2[:, 0::2], A1_2[:, 1::2], ab1_2,
        A2_2[:, 0::2], A2_2[:, 1::2], ab2_2)
    e2, spart2 = att2(a1v2.reshape(-1), a2v2.reshape(-1), pck)
    r2 = _sumrecip(spart2.reshape(2 * _NW, N))
    out2 = agg2(fp2.reshape(-1), e2, pck, r2.reshape(-1))

    return _finish(out2.reshape(HF2, N))


# speedup vs baseline: 1.3134x; 1.0383x over previous
import functools

import jax
import jax.numpy as jnp
from jax import lax
from jax.experimental import pallas as pl
from jax.experimental.pallas import tpu as pltpu
from jax.experimental.pallas import tpu_sc as plsc

_NC = 2
_NS = 16
_NW = _NC * _NS

_BN = 2048


def _mesh():
    return plsc.VectorSubcoreMesh(core_axis_name="c", subcore_axis_name="s")


def _sc_params():
    return pltpu.CompilerParams(needs_layout_passes=False)


def _pack_pair(fe, fo):
    ue = lax.bitcast_convert_type(fe.astype(jnp.bfloat16), jnp.uint16)
    uo = lax.bitcast_convert_type(fo.astype(jnp.bfloat16), jnp.uint16)
    w = (uo.astype(jnp.uint32) << 16) | ue.astype(jnp.uint32)
    return lax.bitcast_convert_type(w, jnp.int32)


def _dense1(x, We, Wo, be, bo, A1e, A1o, ab1, A2e, A2o, ab2):
    N, DIN = x.shape
    HFH = We.shape[0]
    grid = (pl.cdiv(N, _BN),)

    def body(x_ref, we_ref, wo_ref, be_ref, bo_ref,
             a1e_ref, a1o_ref, ab1_ref, a2e_ref, a2o_ref, ab2_ref,
             f_out, a1_out, a2_out):
        x = x_ref[...]
        fe = lax.dot_general(we_ref[...], x, (((1,), (1,)), ((), ())),
                             preferred_element_type=jnp.float32) + be_ref[:, 0:1]
        fo = lax.dot_general(wo_ref[...], x, (((1,), (1,)), ((), ())),
                             preferred_element_type=jnp.float32) + bo_ref[:, 0:1]
        f_out[...] = _pack_pair(fe, fo)
        a1_out[...] = (
            lax.dot_general(a1e_ref[...], fe, (((1,), (0,)), ((), ())),
                            preferred_element_type=jnp.float32)
            + lax.dot_general(a1o_ref[...], fo, (((1,), (0,)), ((), ())),
                              preferred_element_type=jnp.float32)
            + ab1_ref[:, 0:1])
        a2_out[...] = (
            lax.dot_general(a2e_ref[...], fe, (((1,), (0,)), ((), ())),
                            preferred_element_type=jnp.float32)
            + lax.dot_general(a2o_ref[...], fo, (((1,), (0,)), ((), ())),
                              preferred_element_type=jnp.float32)
            + ab2_ref[:, 0:1])

    return pl.pallas_call(
        body,
        grid=grid,
        in_specs=[
            pl.BlockSpec((_BN, DIN), lambda i: (i, 0)),
            pl.BlockSpec((HFH, DIN), lambda i: (0, 0)),
            pl.BlockSpec((HFH, DIN), lambda i: (0, 0)),
            pl.BlockSpec((HFH, 128), lambda i: (0, 0)),
            pl.BlockSpec((HFH, 128), lambda i: (0, 0)),
            pl.BlockSpec((8, HFH), lambda i: (0, 0)),
            pl.BlockSpec((8, HFH), lambda i: (0, 0)),
            pl.BlockSpec((8, 128), lambda i: (0, 0)),
            pl.BlockSpec((8, HFH), lambda i: (0, 0)),
            pl.BlockSpec((8, HFH), lambda i: (0, 0)),
            pl.BlockSpec((8, 128), lambda i: (0, 0)),
        ],
        out_specs=[
            pl.BlockSpec((HFH, _BN), lambda i: (0, i)),
            pl.BlockSpec((8, _BN), lambda i: (0, i)),
            pl.BlockSpec((8, _BN), lambda i: (0, i)),
        ],
        out_shape=[
            jax.ShapeDtypeStruct((HFH, N), jnp.int32),
            jax.ShapeDtypeStruct((8, N), jnp.float32),
            jax.ShapeDtypeStruct((8, N), jnp.float32),
        ],
    )(x, We, Wo, be, bo, A1e, A1o, ab1, A2e, A2o, ab2)


def _dense2(xP, We, Wo, be, bo, A1e, A1o, ab1, A2e, A2o, ab2):
    F = xP.shape[0] // 2
    N = xP.shape[1]
    HFH = We.shape[0]
    grid = (pl.cdiv(N, _BN),)

    def body(x0_ref, x1_ref, we_ref, wo_ref, be_ref, bo_ref,
             a1e_ref, a1o_ref, ab1_ref, a2e_ref, a2o_ref, ab2_ref,
             f_out, a1_out, a2_out):
        x2 = jnp.maximum(x0_ref[...] + x1_ref[...], 0.0)
        fe = lax.dot_general(we_ref[...], x2, (((1,), (0,)), ((), ())),
                             preferred_element_type=jnp.float32) + be_ref[:, 0:1]
        fo = lax.dot_general(wo_ref[...], x2, (((1,), (0,)), ((), ())),
                             preferred_element_type=jnp.float32) + bo_ref[:, 0:1]
        f_out[...] = _pack_pair(fe, fo)
        a1_out[...] = (
            lax.dot_general(a1e_ref[...], fe, (((1,), (0,)), ((), ())),
                            preferred_element_type=jnp.float32)
            + lax.dot_general(a1o_ref[...], fo, (((1,), (0,)), ((), ())),
                              preferred_element_type=jnp.float32)
            + ab1_ref[:, 0:1])
        a2_out[...] = (
            lax.dot_general(a2e_ref[...], fe, (((1,), (0,)), ((), ())),
                            preferred_element_type=jnp.float32)
            + lax.dot_general(a2o_ref[...], fo, (((1,), (0,)), ((), ())),
                              preferred_element_type=jnp.float32)
            + ab2_ref[:, 0:1])

    return pl.pallas_call(
        body,
        grid=grid,
        in_specs=[
            pl.BlockSpec((F, _BN), lambda i: (0, i)),
            pl.BlockSpec((F, _BN), lambda i: (1, i)),
            pl.BlockSpec((HFH, F), lambda i: (0, 0)),
            pl.BlockSpec((HFH, F), lambda i: (0, 0)),
            pl.BlockSpec((HFH, 128), lambda i: (0, 0)),
            pl.BlockSpec((HFH, 128), lambda i: (0, 0)),
            pl.BlockSpec((8, HFH), lambda i: (0, 0)),
            pl.BlockSpec((8, HFH), lambda i: (0, 0)),
            pl.BlockSpec((8, 128), lambda i: (0, 0)),
            pl.BlockSpec((8, HFH), lambda i: (0, 0)),
            pl.BlockSpec((8, HFH), lambda i: (0, 0)),
            pl.BlockSpec((8, 128), lambda i: (0, 0)),
        ],
        out_specs=[
            pl.BlockSpec((HFH, _BN), lambda i: (0, i)),
            pl.BlockSpec((8, _BN), lambda i: (0, i)),
            pl.BlockSpec((8, _BN), lambda i: (0, i)),
        ],
        out_shape=[
            jax.ShapeDtypeStruct((HFH, N), jnp.int32),
            jax.ShapeDtypeStruct((8, N), jnp.float32),
            jax.ShapeDtypeStruct((8, N), jnp.float32),
        ],
    )(xP, xP, We, Wo, be, bo, A1e, A1o, ab1, A2e, A2o, ab2)


def _sumrecip(spart):
    R, N = spart.shape
    half = R // 2
    grid = (pl.cdiv(N, _BN),)

    def body(s_ref, r_out):
        s = s_ref[...]
        s0 = jnp.sum(s[:half], axis=0, keepdims=True)
        s1 = jnp.sum(s[half:], axis=0, keepdims=True)
        r_out[0:1, :] = 1.0 / s0
        r_out[1:2, :] = 1.0 / s1
        r_out[2:8, :] = jnp.zeros_like(r_out[2:8, :])

    return pl.pallas_call(
        body,
        grid=grid,
        in_specs=[pl.BlockSpec((R, _BN), lambda i: (0, i))],
        out_specs=pl.BlockSpec((8, _BN), lambda i: (0, i)),
        out_shape=jax.ShapeDtypeStruct((8, N), jnp.float32),
    )(spart)


def _finish(pT):
    F, N = pT.shape
    BNf = 512
    grid = (pl.cdiv(N, BNf),)

    def body(p_ref, o_ref):
        y = jnp.maximum(p_ref[...], 0.0)
        o_ref[...] = y.T

    return pl.pallas_call(
        body,
        grid=grid,
        in_specs=[pl.BlockSpec((F, BNf), lambda i: (0, i))],
        out_specs=pl.BlockSpec((BNf, F), lambda i: (i, 0)),
        out_shape=jax.ShapeDtypeStruct((N, F), jnp.float32),
    )(pT)


def _make_att(N, E, from_packed):
    CH = E // _NW
    twoN = 2 * N

    out_type = [
        jax.ShapeDtypeStruct((2 * E,), jnp.float32),
        jax.ShapeDtypeStruct((2 * _NW * N,), jnp.float32),
    ]
    scratch = [
        pltpu.VMEM((twoN,), jnp.float32),
        pltpu.VMEM((twoN,), jnp.float32),
        pltpu.VMEM((twoN,), jnp.float32),
        pltpu.VMEM((2 * CH,), jnp.float32),
        pltpu.VMEM((CH,), jnp.int32),
    ]
    if not from_packed:
        out_type.append(jax.ShapeDtypeStruct((E,), jnp.int32))
        scratch.append(pltpu.VMEM((CH,), jnp.int32))
        scratch.append(pltpu.VMEM((CH,), jnp.int32))

    def body(av1_ref, av2_ref, edges_ref, *refs):
        if from_packed:
            e_out, s_out, a1t, a2t, s_loc, e_v, pck_v = refs
        else:
            (e_out, s_out, pck_out,
             a1t, a2t, s_loc, e_v, pck_v, src_v, dst_v) = refs
        wid = lax.axis_index("s") * _NC + lax.axis_index("c")
        base = wid * CH

        pltpu.sync_copy(av1_ref.at[pl.ds(0, twoN)], a1t)
        pltpu.sync_copy(av2_ref.at[pl.ds(0, twoN)], a2t)
        if from_packed:
            pltpu.sync_copy(edges_ref.at[pl.ds(base, CH)], pck_v)
        else:
            pltpu.sync_copy(edges_ref.at[pl.ds(base, CH)], src_v)
            pltpu.sync_copy(edges_ref.at[pl.ds(E + base, CH)], dst_v)

        zero16 = jnp.zeros((16,), jnp.float32)

        @plsc.parallel_loop(0, twoN, 16, unroll=8)
        def _zero(off):
            s_loc[pl.ds(off, 16)] = zero16

        @plsc.parallel_loop(0, CH, 16, unroll=8)
        def _edges(off):
            if from_packed:
                pk = pck_v[pl.ds(off, 16)]
                s16 = pk >> 14
                d16 = pk & 16383
            else:
                s16 = src_v[pl.ds(off, 16)]
                d16 = dst_v[pl.ds(off, 16)]
                pck_v[pl.ds(off, 16)] = (s16 << 14) | d16
            for h in range(2):
                a1v = plsc.load_gather(a1t, [s16 + (h * N)])
                a2v = plsc.load_gather(a2t, [d16 + (h * N)])
                v = a1v + a2v
                v = jnp.where(v > 0.0, v, 0.01 * v)
                ev = jnp.exp(v)
                e_v[pl.ds(h * CH + off, 16)] = ev
                plsc.addupdate_scatter(s_loc, [s16 + (h * N)], ev)

        pltpu.sync_copy(e_v.at[pl.ds(0, CH)], e_out.at[pl.ds(base, CH)])
        pltpu.sync_copy(e_v.at[pl.ds(CH, CH)], e_out.at[pl.ds(E + base, CH)])
        pltpu.sync_copy(s_loc.at[pl.ds(0, N)],
                        s_out.at[pl.ds(wid * N, N)])
        pltpu.sync_copy(s_loc.at[pl.ds(N, N)],
                        s_out.at[pl.ds(_NW * N + wid * N, N)])
        if not from_packed:
            pltpu.sync_copy(pck_v, pck_out.at[pl.ds(base, CH)])

    return pl.kernel(body, out_type=tuple(out_type), mesh=_mesh(),
                     compiler_params=_sc_params(), scratch_types=scratch)


def _make_agg(N, E, F_all, CH2, edge_split=False):
    NSC = 2 if edge_split else 1
    TILES = _NS if edge_split else _NW
    K = F_all // TILES
    KH = K // 2
    half = F_all // 2
    ESC = E // NSC

    NCH = ESC // CH2

    scratch = [
        pltpu.VMEM((KH * N,), jnp.int32),
        pltpu.VMEM((N,), jnp.float32),
        pltpu.VMEM((K * N,), jnp.float32),
        pltpu.VMEM((CH2,), jnp.int32),
        pltpu.VMEM((CH2,), jnp.int32),
        pltpu.VMEM((CH2,), jnp.float32),
        pltpu.VMEM((CH2,), jnp.float32),
        pltpu.SemaphoreType.DMA,
        pltpu.SemaphoreType.DMA,
    ]

    def body(fT_ref, e_ref, pck_ref, r_ref, out_ref,
             f_t, r_t, out_t, pck_b0, pck_b1, e_b0, e_b1, sem_p, sem_e):
        sid = lax.axis_index("s")
        cid = lax.axis_index("c")
        if edge_split:
            tid = sid
            ebase = cid * ESC
        else:
            tid = sid * _NC + cid
            ebase = 0
        c0 = tid * K
        h = c0 // half
        pck_bufs = (pck_b0, pck_b1)
        e_bufs = (e_b0, e_b1)

        pltpu.sync_copy(fT_ref.at[pl.ds(tid * (KH * N), KH * N)], f_t)
        pltpu.sync_copy(r_ref.at[pl.ds(h * N, N)], r_t)

        zero16 = jnp.zeros((16,), jnp.float32)

        @plsc.parallel_loop(0, K * N, 16, unroll=8)
        def _zero(off):
            out_t[pl.ds(off, 16)] = zero16

        def start(ci, b):
            pltpu.async_copy(pck_ref.at[pl.ds(ebase + ci * CH2, CH2)],
                             pck_bufs[b], sem_p)
            pltpu.async_copy(e_ref.at[pl.ds(h * E + ebase + ci * CH2, CH2)],
                             e_bufs[b], sem_e)

        def wait(ci, b):
            pltpu.make_async_copy(pck_ref.at[pl.ds(ebase + ci * CH2, CH2)],
                                  pck_bufs[b], sem_p).wait()
            pltpu.make_async_copy(e_ref.at[pl.ds(h * E + ebase + ci * CH2, CH2)],
                                  e_bufs[b], sem_e).wait()

        start(0, 0)

        def pair_body(cp, _):
            for b in range(2):
                ci = cp * 2 + b

                @pl.when(ci + 1 < NCH)
                def _():
                    start(ci + 1, 1 - b)

                wait(ci, b)
                pck_b = pck_bufs[b]
                e_b = e_bufs[b]

                @plsc.parallel_loop(0, CH2, 16, unroll=8)
                def _edges(off):
                    pk = pck_b[pl.ds(off, 16)]
                    s16 = pk >> 14
                    d16 = pk & 16383
                    ev = e_b[pl.ds(off, 16)]
                    rv = plsc.load_gather(r_t, [s16])
                    att = ev * rv
                    for cp in range(KH):
                        w16 = plsc.load_gather(f_t, [d16 + (cp * N)])
                        flo, fhi = plsc.unpack(
                            plsc.bitcast(w16, jnp.bfloat16),
                            format=plsc.PackFormat.INTERLEAVED)
                        plsc.addupdate_scatter(
                            out_t, [s16 + ((2 * cp) * N)], att * flo)
                        plsc.addupdate_scatter(
                            out_t, [s16 + ((2 * cp + 1) * N)], att * fhi)

            return 0

        lax.fori_loop(0, NCH // 2, pair_body, 0)

        if edge_split:
            pltpu.sync_copy(
                out_t, out_ref.at[pl.ds((cid * F_all + c0) * N, K * N)])
        else:
            pltpu.sync_copy(out_t, out_ref.at[pl.ds(c0 * N, K * N)])

    return pl.kernel(
        body,
        out_type=jax.ShapeDtypeStruct((NSC * F_all * N,), jnp.float32),
        mesh=_mesh(), compiler_params=_sc_params(),
        scratch_types=scratch)


def _blockdiag(aw):
    H, F = aw.shape
    A = jnp.zeros((8, H * F), jnp.float32)
    for h in range(H):
        A = A.at[h, h * F:(h + 1) * F].set(aw[h])
    return A


def _bcast_col(v):
    return jnp.broadcast_to(v.reshape(-1, 1), (v.size, 128)).astype(jnp.float32)


def kernel(features, edge_index, W1, b1, a1w1, a1b1, a2w1, a2b1,
           W2, b2, a1w2, a1b2, a2w2, a2b2):
    N, DIN = features.shape
    E = edge_index.shape[1]
    H, F1, _ = W1.shape
    F2 = W2.shape[1]
    HF1, HF2 = H * F1, H * F2

    Ws1 = W1.reshape(HF1, DIN)
    Ws2 = W2.reshape(HF2, HF1)
    bs1 = b1.reshape(HF1)
    bs2 = b2.reshape(HF2)
    A1_1, A2_1 = _blockdiag(a1w1), _blockdiag(a2w1)
    A1_2, A2_2 = _blockdiag(a1w2), _blockdiag(a2w2)
    ab1_1 = _bcast_col(jnp.pad(a1b1, (0, 8 - H)))
    ab2_1 = _bcast_col(jnp.pad(a2b1, (0, 8 - H)))
    ab1_2 = _bcast_col(jnp.pad(a1b2, (0, 8 - H)))
    ab2_2 = _bcast_col(jnp.pad(a2b2, (0, 8 - H)))

    att1 = _make_att(N, E, from_packed=False)
    att2 = _make_att(N, E, from_packed=True)
    agg1 = _make_agg(N, E, HF1, 8000, edge_split=True)
    agg2 = _make_agg(N, E, HF2, 8000)

    fp1, a1v1, a2v1 = _dense1(
        features, Ws1[0::2], Ws1[1::2],
        _bcast_col(bs1[0::2]), _bcast_col(bs1[1::2]),
        A1_1[:, 0::2], A1_1[:, 1::2], ab1_1,
        A2_1[:, 0::2], A2_1[:, 1::2], ab2_1)
    e1, spart1, pck = att1(a1v1.reshape(-1), a2v1.reshape(-1),
                           edge_index.reshape(-1))
    r1 = _sumrecip(spart1.reshape(2 * _NW, N))
    out1 = agg1(fp1.reshape(-1), e1, pck, r1.reshape(-1))

    fp2, a1v2, a2v2 = _dense2(
        out1.reshape(2 * HF1, N), Ws2[0::2], Ws2[1::2],
        _bcast_col(bs2[0::2]), _bcast_col(bs2[1::2]),
        A1_2[:, 0::2], A1_2[:, 1::2], ab1_2,
        A2_2[:, 0::2], A2_2[:, 1::2], ab2_2)
    e2, spart2 = att2(a1v2.reshape(-1), a2v2.reshape(-1), pck)
    r2 = _sumrecip(spart2.reshape(2 * _NW, N))
    out2 = agg2(fp2.reshape(-1), e2, pck, r2.reshape(-1))

    return _finish(out2.reshape(HF2, N))


# speedup vs baseline: 1.3635x; 1.0382x over previous
import functools

import jax
import jax.numpy as jnp
from jax import lax
from jax.experimental import pallas as pl
from jax.experimental.pallas import tpu as pltpu
from jax.experimental.pallas import tpu_sc as plsc

_NC = 2
_NS = 16
_NW = _NC * _NS

_BN = 2048


def _mesh():
    return plsc.VectorSubcoreMesh(core_axis_name="c", subcore_axis_name="s")


def _sc_params():
    return pltpu.CompilerParams(needs_layout_passes=False)


def _pack_pair(fe, fo):
    ue = lax.bitcast_convert_type(fe.astype(jnp.bfloat16), jnp.uint16)
    uo = lax.bitcast_convert_type(fo.astype(jnp.bfloat16), jnp.uint16)
    w = (uo.astype(jnp.uint32) << 16) | ue.astype(jnp.uint32)
    return lax.bitcast_convert_type(w, jnp.int32)


def _dense1(x, We, Wo, be, bo, A1e, A1o, ab1, A2e, A2o, ab2):
    N, DIN = x.shape
    HFH = We.shape[0]
    grid = (pl.cdiv(N, _BN),)

    def body(x_ref, we_ref, wo_ref, be_ref, bo_ref,
             a1e_ref, a1o_ref, ab1_ref, a2e_ref, a2o_ref, ab2_ref,
             f_out, a1_out, a2_out):
        x = x_ref[...]
        fe = lax.dot_general(we_ref[...], x, (((1,), (1,)), ((), ())),
                             preferred_element_type=jnp.float32) + be_ref[:, 0:1]
        fo = lax.dot_general(wo_ref[...], x, (((1,), (1,)), ((), ())),
                             preferred_element_type=jnp.float32) + bo_ref[:, 0:1]
        f_out[...] = _pack_pair(fe, fo)
        a1_out[...] = (
            lax.dot_general(a1e_ref[...], fe, (((1,), (0,)), ((), ())),
                            preferred_element_type=jnp.float32)
            + lax.dot_general(a1o_ref[...], fo, (((1,), (0,)), ((), ())),
                              preferred_element_type=jnp.float32)
            + ab1_ref[:, 0:1])
        a2_out[...] = (
            lax.dot_general(a2e_ref[...], fe, (((1,), (0,)), ((), ())),
                            preferred_element_type=jnp.float32)
            + lax.dot_general(a2o_ref[...], fo, (((1,), (0,)), ((), ())),
                              preferred_element_type=jnp.float32)
            + ab2_ref[:, 0:1])

    return pl.pallas_call(
        body,
        grid=grid,
        in_specs=[
            pl.BlockSpec((_BN, DIN), lambda i: (i, 0)),
            pl.BlockSpec((HFH, DIN), lambda i: (0, 0)),
            pl.BlockSpec((HFH, DIN), lambda i: (0, 0)),
            pl.BlockSpec((HFH, 128), lambda i: (0, 0)),
            pl.BlockSpec((HFH, 128), lambda i: (0, 0)),
            pl.BlockSpec((8, HFH), lambda i: (0, 0)),
            pl.BlockSpec((8, HFH), lambda i: (0, 0)),
            pl.BlockSpec((8, 128), lambda i: (0, 0)),
            pl.BlockSpec((8, HFH), lambda i: (0, 0)),
            pl.BlockSpec((8, HFH), lambda i: (0, 0)),
            pl.BlockSpec((8, 128), lambda i: (0, 0)),
        ],
        out_specs=[
            pl.BlockSpec((HFH, _BN), lambda i: (0, i)),
            pl.BlockSpec((8, _BN), lambda i: (0, i)),
            pl.BlockSpec((8, _BN), lambda i: (0, i)),
        ],
        out_shape=[
            jax.ShapeDtypeStruct((HFH, N), jnp.int32),
            jax.ShapeDtypeStruct((8, N), jnp.float32),
            jax.ShapeDtypeStruct((8, N), jnp.float32),
        ],
    )(x, We, Wo, be, bo, A1e, A1o, ab1, A2e, A2o, ab2)


def _dense2(xP, We, Wo, be, bo, A1e, A1o, ab1, A2e, A2o, ab2):
    F = xP.shape[0] // 2
    N = xP.shape[1]
    HFH = We.shape[0]
    grid = (pl.cdiv(N, _BN),)

    def body(x0_ref, x1_ref, we_ref, wo_ref, be_ref, bo_ref,
             a1e_ref, a1o_ref, ab1_ref, a2e_ref, a2o_ref, ab2_ref,
             f_out, a1_out, a2_out):
        x2 = jnp.maximum(x0_ref[...] + x1_ref[...], 0.0)
        fe = lax.dot_general(we_ref[...], x2, (((1,), (0,)), ((), ())),
                             preferred_element_type=jnp.float32) + be_ref[:, 0:1]
        fo = lax.dot_general(wo_ref[...], x2, (((1,), (0,)), ((), ())),
                             preferred_element_type=jnp.float32) + bo_ref[:, 0:1]
        f_out[...] = _pack_pair(fe, fo)
        a1_out[...] = (
            lax.dot_general(a1e_ref[...], fe, (((1,), (0,)), ((), ())),
                            preferred_element_type=jnp.float32)
            + lax.dot_general(a1o_ref[...], fo, (((1,), (0,)), ((), ())),
                              preferred_element_type=jnp.float32)
            + ab1_ref[:, 0:1])
        a2_out[...] = (
            lax.dot_general(a2e_ref[...], fe, (((1,), (0,)), ((), ())),
                            preferred_element_type=jnp.float32)
            + lax.dot_general(a2o_ref[...], fo, (((1,), (0,)), ((), ())),
                              preferred_element_type=jnp.float32)
            + ab2_ref[:, 0:1])

    return pl.pallas_call(
        body,
        grid=grid,
        in_specs=[
            pl.BlockSpec((F, _BN), lambda i: (0, i)),
            pl.BlockSpec((F, _BN), lambda i: (1, i)),
            pl.BlockSpec((HFH, F), lambda i: (0, 0)),
            pl.BlockSpec((HFH, F), lambda i: (0, 0)),
            pl.BlockSpec((HFH, 128), lambda i: (0, 0)),
            pl.BlockSpec((HFH, 128), lambda i: (0, 0)),
            pl.BlockSpec((8, HFH), lambda i: (0, 0)),
            pl.BlockSpec((8, HFH), lambda i: (0, 0)),
            pl.BlockSpec((8, 128), lambda i: (0, 0)),
            pl.BlockSpec((8, HFH), lambda i: (0, 0)),
            pl.BlockSpec((8, HFH), lambda i: (0, 0)),
            pl.BlockSpec((8, 128), lambda i: (0, 0)),
        ],
        out_specs=[
            pl.BlockSpec((HFH, _BN), lambda i: (0, i)),
            pl.BlockSpec((8, _BN), lambda i: (0, i)),
            pl.BlockSpec((8, _BN), lambda i: (0, i)),
        ],
        out_shape=[
            jax.ShapeDtypeStruct((HFH, N), jnp.int32),
            jax.ShapeDtypeStruct((8, N), jnp.float32),
            jax.ShapeDtypeStruct((8, N), jnp.float32),
        ],
    )(xP, xP, We, Wo, be, bo, A1e, A1o, ab1, A2e, A2o, ab2)


def _sumrecip(spart):
    R, N = spart.shape
    half = R // 2
    grid = (pl.cdiv(N, _BN),)

    def body(s_ref, r_out):
        s = s_ref[...]
        s0 = jnp.sum(s[:half], axis=0, keepdims=True)
        s1 = jnp.sum(s[half:], axis=0, keepdims=True)
        r_out[0:1, :] = 1.0 / s0
        r_out[1:2, :] = 1.0 / s1
        r_out[2:8, :] = jnp.zeros_like(r_out[2:8, :])

    return pl.pallas_call(
        body,
        grid=grid,
        in_specs=[pl.BlockSpec((R, _BN), lambda i: (0, i))],
        out_specs=pl.BlockSpec((8, _BN), lambda i: (0, i)),
        out_shape=jax.ShapeDtypeStruct((8, N), jnp.float32),
    )(spart)


def _finish(pT):
    F, N = pT.shape
    BNf = 512
    grid = (pl.cdiv(N, BNf),)

    def body(p_ref, o_ref):
        y = jnp.maximum(p_ref[...], 0.0)
        o_ref[...] = y.T

    return pl.pallas_call(
        body,
        grid=grid,
        in_specs=[pl.BlockSpec((F, BNf), lambda i: (0, i))],
        out_specs=pl.BlockSpec((BNf, F), lambda i: (i, 0)),
        out_shape=jax.ShapeDtypeStruct((N, F), jnp.float32),
    )(pT)


def _make_att(N, E, from_packed):
    CH = E // _NW
    twoN = 2 * N

    out_type = [
        jax.ShapeDtypeStruct((2 * E,), jnp.float32),
        jax.ShapeDtypeStruct((2 * _NW * N,), jnp.float32),
    ]
    scratch = [
        pltpu.VMEM((twoN,), jnp.float32),
        pltpu.VMEM((twoN,), jnp.float32),
        pltpu.VMEM((twoN,), jnp.float32),
        pltpu.VMEM((2 * CH,), jnp.float32),
        pltpu.VMEM((CH,), jnp.int32),
    ]
    if not from_packed:
        out_type.append(jax.ShapeDtypeStruct((E,), jnp.int32))
        scratch.append(pltpu.VMEM((CH,), jnp.int32))
        scratch.append(pltpu.VMEM((CH,), jnp.int32))

    def body(av1_ref, av2_ref, edges_ref, *refs):
        if from_packed:
            e_out, s_out, a1t, a2t, s_loc, e_v, pck_v = refs
        else:
            (e_out, s_out, pck_out,
             a1t, a2t, s_loc, e_v, pck_v, src_v, dst_v) = refs
        wid = lax.axis_index("s") * _NC + lax.axis_index("c")
        base = wid * CH

        pltpu.sync_copy(av1_ref.at[pl.ds(0, twoN)], a1t)
        pltpu.sync_copy(av2_ref.at[pl.ds(0, twoN)], a2t)
        if from_packed:
            pltpu.sync_copy(edges_ref.at[pl.ds(base, CH)], pck_v)
        else:
            pltpu.sync_copy(edges_ref.at[pl.ds(base, CH)], src_v)
            pltpu.sync_copy(edges_ref.at[pl.ds(E + base, CH)], dst_v)

        zero16 = jnp.zeros((16,), jnp.float32)

        @plsc.parallel_loop(0, twoN, 16, unroll=8)
        def _zero(off):
            s_loc[pl.ds(off, 16)] = zero16

        @plsc.parallel_loop(0, CH, 16, unroll=8)
        def _edges(off):
            if from_packed:
                pk = pck_v[pl.ds(off, 16)]
                s16 = pk >> 14
                d16 = pk & 16383
            else:
                s16 = src_v[pl.ds(off, 16)]
                d16 = dst_v[pl.ds(off, 16)]
                pck_v[pl.ds(off, 16)] = (s16 << 14) | d16
            for h in range(2):
                a1v = plsc.load_gather(a1t, [s16 + (h * N)])
                a2v = plsc.load_gather(a2t, [d16 + (h * N)])
                v = a1v + a2v
                v = jnp.where(v > 0.0, v, 0.01 * v)
                ev = jnp.exp(v)
                e_v[pl.ds(h * CH + off, 16)] = ev
                plsc.addupdate_scatter(s_loc, [s16 + (h * N)], ev)

        pltpu.sync_copy(e_v.at[pl.ds(0, CH)], e_out.at[pl.ds(base, CH)])
        pltpu.sync_copy(e_v.at[pl.ds(CH, CH)], e_out.at[pl.ds(E + base, CH)])
        pltpu.sync_copy(s_loc.at[pl.ds(0, N)],
                        s_out.at[pl.ds(wid * N, N)])
        pltpu.sync_copy(s_loc.at[pl.ds(N, N)],
                        s_out.at[pl.ds(_NW * N + wid * N, N)])
        if not from_packed:
            pltpu.sync_copy(pck_v, pck_out.at[pl.ds(base, CH)])

    return pl.kernel(body, out_type=tuple(out_type), mesh=_mesh(),
                     compiler_params=_sc_params(), scratch_types=scratch)


def _make_agg(N, E, F_all, CH2, edge_split=False):
    NSC = 2 if edge_split else 1
    TILES = _NS if edge_split else _NW
    K = F_all // TILES
    KH = K // 2
    half = F_all // 2
    ESC = E // NSC

    NCH = ESC // CH2

    scratch = [
        pltpu.VMEM((KH * N,), jnp.int32),
        pltpu.VMEM((N,), jnp.float32),
        pltpu.VMEM((K * N,), jnp.float32),
        pltpu.VMEM((CH2,), jnp.int32),
        pltpu.VMEM((CH2,), jnp.int32),
        pltpu.VMEM((CH2,), jnp.float32),
        pltpu.VMEM((CH2,), jnp.float32),
        pltpu.SemaphoreType.DMA,
        pltpu.SemaphoreType.DMA,
    ]

    def body(fT_ref, e_ref, pck_ref, r_ref, out_ref,
             f_t, r_t, out_t, pck_b0, pck_b1, e_b0, e_b1, sem_p, sem_e):
        sid = lax.axis_index("s")
        cid = lax.axis_index("c")
        if edge_split:
            tid = sid
            ebase = cid * ESC
        else:
            tid = sid * _NC + cid
            ebase = 0
        c0 = tid * K
        h = c0 // half
        pck_bufs = (pck_b0, pck_b1)
        e_bufs = (e_b0, e_b1)

        pltpu.sync_copy(fT_ref.at[pl.ds(tid * (KH * N), KH * N)], f_t)
        pltpu.sync_copy(r_ref.at[pl.ds(h * N, N)], r_t)

        zero16 = jnp.zeros((16,), jnp.float32)

        @plsc.parallel_loop(0, K * N, 16, unroll=8)
        def _zero(off):
            out_t[pl.ds(off, 16)] = zero16

        def start(ci, b):
            pltpu.async_copy(pck_ref.at[pl.ds(ebase + ci * CH2, CH2)],
                             pck_bufs[b], sem_p)
            pltpu.async_copy(e_ref.at[pl.ds(h * E + ebase + ci * CH2, CH2)],
                             e_bufs[b], sem_e)

        def wait(ci, b):
            pltpu.make_async_copy(pck_ref.at[pl.ds(ebase + ci * CH2, CH2)],
                                  pck_bufs[b], sem_p).wait()
            pltpu.make_async_copy(e_ref.at[pl.ds(h * E + ebase + ci * CH2, CH2)],
                                  e_bufs[b], sem_e).wait()

        start(0, 0)

        def pair_body(cp, _):
            for b in range(2):
                ci = cp * 2 + b

                @pl.when(ci + 1 < NCH)
                def _():
                    start(ci + 1, 1 - b)

                wait(ci, b)
                pck_b = pck_bufs[b]
                e_b = e_bufs[b]

                @plsc.parallel_loop(0, CH2, 16, unroll=8)
                def _edges(off):
                    pk = pck_b[pl.ds(off, 16)]
                    s16 = pk >> 14
                    d16 = pk & 16383
                    ev = e_b[pl.ds(off, 16)]
                    for cp in range(KH):
                        w16 = plsc.load_gather(f_t, [d16 + (cp * N)])
                        flo, fhi = plsc.unpack(
                            plsc.bitcast(w16, jnp.bfloat16),
                            format=plsc.PackFormat.INTERLEAVED)
                        plsc.addupdate_scatter(
                            out_t, [s16 + ((2 * cp) * N)], ev * flo)
                        plsc.addupdate_scatter(
                            out_t, [s16 + ((2 * cp + 1) * N)], ev * fhi)

            return 0

        lax.fori_loop(0, NCH // 2, pair_body, 0)

        @plsc.parallel_loop(0, N, 16, unroll=8)
        def _scale(off):
            rv = r_t[pl.ds(off, 16)]
            for c in range(K):
                out_t[pl.ds(c * N + off, 16)] = (
                    out_t[pl.ds(c * N + off, 16)] * rv)

        if edge_split:
            pltpu.sync_copy(
                out_t, out_ref.at[pl.ds((cid * F_all + c0) * N, K * N)])
        else:
            pltpu.sync_copy(out_t, out_ref.at[pl.ds(c0 * N, K * N)])

    return pl.kernel(
        body,
        out_type=jax.ShapeDtypeStruct((NSC * F_all * N,), jnp.float32),
        mesh=_mesh(), compiler_params=_sc_params(),
        scratch_types=scratch)


def _blockdiag(aw):
    H, F = aw.shape
    A = jnp.zeros((8, H * F), jnp.float32)
    for h in range(H):
        A = A.at[h, h * F:(h + 1) * F].set(aw[h])
    return A


def _bcast_col(v):
    return jnp.broadcast_to(v.reshape(-1, 1), (v.size, 128)).astype(jnp.float32)


def kernel(features, edge_index, W1, b1, a1w1, a1b1, a2w1, a2b1,
           W2, b2, a1w2, a1b2, a2w2, a2b2):
    N, DIN = features.shape
    E = edge_index.shape[1]
    H, F1, _ = W1.shape
    F2 = W2.shape[1]
    HF1, HF2 = H * F1, H * F2

    Ws1 = W1.reshape(HF1, DIN)
    Ws2 = W2.reshape(HF2, HF1)
    bs1 = b1.reshape(HF1)
    bs2 = b2.reshape(HF2)
    A1_1, A2_1 = _blockdiag(a1w1), _blockdiag(a2w1)
    A1_2, A2_2 = _blockdiag(a1w2), _blockdiag(a2w2)
    ab1_1 = _bcast_col(jnp.pad(a1b1, (0, 8 - H)))
    ab2_1 = _bcast_col(jnp.pad(a2b1, (0, 8 - H)))
    ab1_2 = _bcast_col(jnp.pad(a1b2, (0, 8 - H)))
    ab2_2 = _bcast_col(jnp.pad(a2b2, (0, 8 - H)))

    att1 = _make_att(N, E, from_packed=False)
    att2 = _make_att(N, E, from_packed=True)
    agg1 = _make_agg(N, E, HF1, 8000, edge_split=True)
    agg2 = _make_agg(N, E, HF2, 8000)

    fp1, a1v1, a2v1 = _dense1(
        features, Ws1[0::2], Ws1[1::2],
        _bcast_col(bs1[0::2]), _bcast_col(bs1[1::2]),
        A1_1[:, 0::2], A1_1[:, 1::2], ab1_1,
        A2_1[:, 0::2], A2_1[:, 1::2], ab2_1)
    e1, spart1, pck = att1(a1v1.reshape(-1), a2v1.reshape(-1),
                           edge_index.reshape(-1))
    r1 = _sumrecip(spart1.reshape(2 * _NW, N))
    out1 = agg1(fp1.reshape(-1), e1, pck, r1.reshape(-1))

    fp2, a1v2, a2v2 = _dense2(
        out1.reshape(2 * HF1, N), Ws2[0::2], Ws2[1::2],
        _bcast_col(bs2[0::2]), _bcast_col(bs2[1::2]),
        A1_2[:, 0::2], A1_2[:, 1::2], ab1_2,
        A2_2[:, 0::2], A2_2[:, 1::2], ab2_2)
    e2, spart2 = att2(a1v2.reshape(-1), a2v2.reshape(-1), pck)
    r2 = _sumrecip(spart2.reshape(2 * _NW, N))
    out2 = agg2(fp2.reshape(-1), e2, pck, r2.reshape(-1))

    return _finish(out2.reshape(HF2, N))


# speedup vs baseline: 1.4363x; 1.0534x over previous
import functools

import jax
import jax.numpy as jnp
from jax import lax
from jax.experimental import pallas as pl
from jax.experimental.pallas import tpu as pltpu
from jax.experimental.pallas import tpu_sc as plsc

_NC = 2
_NS = 16
_NW = _NC * _NS

_BN = 2048


def _mesh():
    return plsc.VectorSubcoreMesh(core_axis_name="c", subcore_axis_name="s")


def _sc_params():
    return pltpu.CompilerParams(needs_layout_passes=False)


def _pack_pair(fe, fo):
    ue = lax.bitcast_convert_type(fe.astype(jnp.bfloat16), jnp.uint16)
    uo = lax.bitcast_convert_type(fo.astype(jnp.bfloat16), jnp.uint16)
    w = (uo.astype(jnp.uint32) << 16) | ue.astype(jnp.uint32)
    return lax.bitcast_convert_type(w, jnp.int32)


def _dense1(x, We, Wo, be, bo, A1e, A1o, ab1, A2e, A2o, ab2):
    N, DIN = x.shape
    HFH = We.shape[0]
    grid = (pl.cdiv(N, _BN),)

    def body(x_ref, we_ref, wo_ref, be_ref, bo_ref,
             a1e_ref, a1o_ref, ab1_ref, a2e_ref, a2o_ref, ab2_ref,
             f_out, a1_out, a2_out):
        x = x_ref[...]
        fe = lax.dot_general(we_ref[...], x, (((1,), (1,)), ((), ())),
                             preferred_element_type=jnp.float32) + be_ref[:, 0:1]
        fo = lax.dot_general(wo_ref[...], x, (((1,), (1,)), ((), ())),
                             preferred_element_type=jnp.float32) + bo_ref[:, 0:1]
        f_out[...] = _pack_pair(fe, fo)
        a1_out[...] = (
            lax.dot_general(a1e_ref[...], fe, (((1,), (0,)), ((), ())),
                            preferred_element_type=jnp.float32)
            + lax.dot_general(a1o_ref[...], fo, (((1,), (0,)), ((), ())),
                              preferred_element_type=jnp.float32)
            + ab1_ref[:, 0:1])
        a2_out[...] = (
            lax.dot_general(a2e_ref[...], fe, (((1,), (0,)), ((), ())),
                            preferred_element_type=jnp.float32)
            + lax.dot_general(a2o_ref[...], fo, (((1,), (0,)), ((), ())),
                              preferred_element_type=jnp.float32)
            + ab2_ref[:, 0:1])

    return pl.pallas_call(
        body,
        grid=grid,
        in_specs=[
            pl.BlockSpec((_BN, DIN), lambda i: (i, 0)),
            pl.BlockSpec((HFH, DIN), lambda i: (0, 0)),
            pl.BlockSpec((HFH, DIN), lambda i: (0, 0)),
            pl.BlockSpec((HFH, 128), lambda i: (0, 0)),
            pl.BlockSpec((HFH, 128), lambda i: (0, 0)),
            pl.BlockSpec((8, HFH), lambda i: (0, 0)),
            pl.BlockSpec((8, HFH), lambda i: (0, 0)),
            pl.BlockSpec((8, 128), lambda i: (0, 0)),
            pl.BlockSpec((8, HFH), lambda i: (0, 0)),
            pl.BlockSpec((8, HFH), lambda i: (0, 0)),
            pl.BlockSpec((8, 128), lambda i: (0, 0)),
        ],
        out_specs=[
            pl.BlockSpec((HFH, _BN), lambda i: (0, i)),
            pl.BlockSpec((8, _BN), lambda i: (0, i)),
            pl.BlockSpec((8, _BN), lambda i: (0, i)),
        ],
        out_shape=[
            jax.ShapeDtypeStruct((HFH, N), jnp.int32),
            jax.ShapeDtypeStruct((8, N), jnp.float32),
            jax.ShapeDtypeStruct((8, N), jnp.float32),
        ],
    )(x, We, Wo, be, bo, A1e, A1o, ab1, A2e, A2o, ab2)


def _dense2(xP, spart, We, Wo, be, bo, A1e, A1o, ab1, A2e, A2o, ab2):
    F = xP.shape[0] // 2
    N = xP.shape[1]
    R = spart.shape[0]
    HFH = We.shape[0]
    FH = F // 2
    grid = (pl.cdiv(N, _BN),)

    def body(x0_ref, x1_ref, s_ref, we_ref, wo_ref, be_ref, bo_ref,
             a1e_ref, a1o_ref, ab1_ref, a2e_ref, a2o_ref, ab2_ref,
             f_out, a1_out, a2_out):
        s = s_ref[...]
        inv0 = 1.0 / jnp.maximum(
            jnp.sum(s[:R // 2], axis=0, keepdims=True), 1e-30)
        inv1 = 1.0 / jnp.maximum(
            jnp.sum(s[R // 2:], axis=0, keepdims=True), 1e-30)
        scale = jnp.concatenate(
            [jnp.broadcast_to(inv0, (FH, inv0.shape[1])),
             jnp.broadcast_to(inv1, (FH, inv1.shape[1]))], axis=0)
        x2 = jnp.maximum(x0_ref[...] + x1_ref[...], 0.0) * scale
        fe = lax.dot_general(we_ref[...], x2, (((1,), (0,)), ((), ())),
                             preferred_element_type=jnp.float32) + be_ref[:, 0:1]
        fo = lax.dot_general(wo_ref[...], x2, (((1,), (0,)), ((), ())),
                             preferred_element_type=jnp.float32) + bo_ref[:, 0:1]
        f_out[...] = _pack_pair(fe, fo)
        a1_out[...] = (
            lax.dot_general(a1e_ref[...], fe, (((1,), (0,)), ((), ())),
                            preferred_element_type=jnp.float32)
            + lax.dot_general(a1o_ref[...], fo, (((1,), (0,)), ((), ())),
                              preferred_element_type=jnp.float32)
            + ab1_ref[:, 0:1])
        a2_out[...] = (
            lax.dot_general(a2e_ref[...], fe, (((1,), (0,)), ((), ())),
                            preferred_element_type=jnp.float32)
            + lax.dot_general(a2o_ref[...], fo, (((1,), (0,)), ((), ())),
                              preferred_element_type=jnp.float32)
            + ab2_ref[:, 0:1])

    return pl.pallas_call(
        body,
        grid=grid,
        in_specs=[
            pl.BlockSpec((F, _BN), lambda i: (0, i)),
            pl.BlockSpec((F, _BN), lambda i: (1, i)),
            pl.BlockSpec((R, _BN), lambda i: (0, i)),
            pl.BlockSpec((HFH, F), lambda i: (0, 0)),
            pl.BlockSpec((HFH, F), lambda i: (0, 0)),
            pl.BlockSpec((HFH, 128), lambda i: (0, 0)),
            pl.BlockSpec((HFH, 128), lambda i: (0, 0)),
            pl.BlockSpec((8, HFH), lambda i: (0, 0)),
            pl.BlockSpec((8, HFH), lambda i: (0, 0)),
            pl.BlockSpec((8, 128), lambda i: (0, 0)),
            pl.BlockSpec((8, HFH), lambda i: (0, 0)),
            pl.BlockSpec((8, HFH), lambda i: (0, 0)),
            pl.BlockSpec((8, 128), lambda i: (0, 0)),
        ],
        out_specs=[
            pl.BlockSpec((HFH, _BN), lambda i: (0, i)),
            pl.BlockSpec((8, _BN), lambda i: (0, i)),
            pl.BlockSpec((8, _BN), lambda i: (0, i)),
        ],
        out_shape=[
            jax.ShapeDtypeStruct((HFH, N), jnp.int32),
            jax.ShapeDtypeStruct((8, N), jnp.float32),
            jax.ShapeDtypeStruct((8, N), jnp.float32),
        ],
    )(xP, xP, spart, We, Wo, be, bo, A1e, A1o, ab1, A2e, A2o, ab2)


def _finish(pT, spart):
    F, N = pT.shape
    R = spart.shape[0]
    FH = F // 2
    BNf = 512
    grid = (pl.cdiv(N, BNf),)

    def body(p_ref, s_ref, o_ref):
        s = s_ref[...]
        inv0 = 1.0 / jnp.maximum(
            jnp.sum(s[:R // 2], axis=0, keepdims=True), 1e-30)
        inv1 = 1.0 / jnp.maximum(
            jnp.sum(s[R // 2:], axis=0, keepdims=True), 1e-30)
        scale = jnp.concatenate(
            [jnp.broadcast_to(inv0, (FH, inv0.shape[1])),
             jnp.broadcast_to(inv1, (FH, inv1.shape[1]))], axis=0)
        y = jnp.maximum(p_ref[...], 0.0) * scale
        o_ref[...] = y.T

    return pl.pallas_call(
        body,
        grid=grid,
        in_specs=[pl.BlockSpec((F, BNf), lambda i: (0, i)),
                  pl.BlockSpec((R, BNf), lambda i: (0, i))],
        out_specs=pl.BlockSpec((BNf, F), lambda i: (i, 0)),
        out_shape=jax.ShapeDtypeStruct((N, F), jnp.float32),
    )(pT, spart)


def _make_att(N, E, from_packed):
    CH = E // _NW
    twoN = 2 * N

    out_type = [
        jax.ShapeDtypeStruct((2 * E,), jnp.float32),
        jax.ShapeDtypeStruct((2 * _NW * N,), jnp.float32),
    ]
    scratch = [
        pltpu.VMEM((twoN,), jnp.float32),
        pltpu.VMEM((twoN,), jnp.float32),
        pltpu.VMEM((twoN,), jnp.float32),
        pltpu.VMEM((2 * CH,), jnp.float32),
        pltpu.VMEM((CH,), jnp.int32),
    ]
    if not from_packed:
        out_type.append(jax.ShapeDtypeStruct((E,), jnp.int32))
        scratch.append(pltpu.VMEM((CH,), jnp.int32))
        scratch.append(pltpu.VMEM((CH,), jnp.int32))

    def body(av1_ref, av2_ref, edges_ref, *refs):
        if from_packed:
            e_out, s_out, a1t, a2t, s_loc, e_v, pck_v = refs
        else:
            (e_out, s_out, pck_out,
             a1t, a2t, s_loc, e_v, pck_v, src_v, dst_v) = refs
        wid = lax.axis_index("s") * _NC + lax.axis_index("c")
        base = wid * CH

        pltpu.sync_copy(av1_ref.at[pl.ds(0, twoN)], a1t)
        pltpu.sync_copy(av2_ref.at[pl.ds(0, twoN)], a2t)
        if from_packed:
            pltpu.sync_copy(edges_ref.at[pl.ds(base, CH)], pck_v)
        else:
            pltpu.sync_copy(edges_ref.at[pl.ds(base, CH)], src_v)
            pltpu.sync_copy(edges_ref.at[pl.ds(E + base, CH)], dst_v)

        zero16 = jnp.zeros((16,), jnp.float32)

        @plsc.parallel_loop(0, twoN, 16, unroll=8)
        def _zero(off):
            s_loc[pl.ds(off, 16)] = zero16

        @plsc.parallel_loop(0, CH, 16, unroll=8)
        def _edges(off):
            if from_packed:
                pk = pck_v[pl.ds(off, 16)]
                s16 = pk >> 14
                d16 = pk & 16383
            else:
                s16 = src_v[pl.ds(off, 16)]
                d16 = dst_v[pl.ds(off, 16)]
                pck_v[pl.ds(off, 16)] = (s16 << 14) | d16
            for h in range(2):
                a1v = plsc.load_gather(a1t, [s16 + (h * N)])
                a2v = plsc.load_gather(a2t, [d16 + (h * N)])
                v = a1v + a2v
                v = jnp.where(v > 0.0, v, 0.01 * v)
                ev = jnp.exp(v)
                e_v[pl.ds(h * CH + off, 16)] = ev
                plsc.addupdate_scatter(s_loc, [s16 + (h * N)], ev)

        pltpu.sync_copy(e_v.at[pl.ds(0, CH)], e_out.at[pl.ds(base, CH)])
        pltpu.sync_copy(e_v.at[pl.ds(CH, CH)], e_out.at[pl.ds(E + base, CH)])
        pltpu.sync_copy(s_loc.at[pl.ds(0, N)],
                        s_out.at[pl.ds(wid * N, N)])
        pltpu.sync_copy(s_loc.at[pl.ds(N, N)],
                        s_out.at[pl.ds(_NW * N + wid * N, N)])
        if not from_packed:
            pltpu.sync_copy(pck_v, pck_out.at[pl.ds(base, CH)])

    return pl.kernel(body, out_type=tuple(out_type), mesh=_mesh(),
                     compiler_params=_sc_params(), scratch_types=scratch)


def _make_agg(N, E, F_all, CH2, edge_split=False):
    NSC = 2 if edge_split else 1
    TILES = _NS if edge_split else _NW
    K = F_all // TILES
    KH = K // 2
    half = F_all // 2
    ESC = E // NSC

    NCH = ESC // CH2

    scratch = [
        pltpu.VMEM((KH * N,), jnp.int32),
        pltpu.VMEM((K * N,), jnp.float32),
        pltpu.VMEM((CH2,), jnp.int32),
        pltpu.VMEM((CH2,), jnp.int32),
        pltpu.VMEM((CH2,), jnp.float32),
        pltpu.VMEM((CH2,), jnp.float32),
        pltpu.SemaphoreType.DMA,
        pltpu.SemaphoreType.DMA,
    ]

    def body(fT_ref, e_ref, pck_ref, out_ref,
             f_t, out_t, pck_b0, pck_b1, e_b0, e_b1, sem_p, sem_e):
        sid = lax.axis_index("s")
        cid = lax.axis_index("c")
        if edge_split:
            tid = sid
            ebase = cid * ESC
        else:
            tid = sid * _NC + cid
            ebase = 0
        c0 = tid * K
        h = c0 // half
        pck_bufs = (pck_b0, pck_b1)
        e_bufs = (e_b0, e_b1)

        pltpu.sync_copy(fT_ref.at[pl.ds(tid * (KH * N), KH * N)], f_t)

        zero16 = jnp.zeros((16,), jnp.float32)

        @plsc.parallel_loop(0, K * N, 16, unroll=8)
        def _zero(off):
            out_t[pl.ds(off, 16)] = zero16

        def start(ci, b):
            pltpu.async_copy(pck_ref.at[pl.ds(ebase + ci * CH2, CH2)],
                             pck_bufs[b], sem_p)
            pltpu.async_copy(e_ref.at[pl.ds(h * E + ebase + ci * CH2, CH2)],
                             e_bufs[b], sem_e)

        def wait(ci, b):
            pltpu.make_async_copy(pck_ref.at[pl.ds(ebase + ci * CH2, CH2)],
                                  pck_bufs[b], sem_p).wait()
            pltpu.make_async_copy(e_ref.at[pl.ds(h * E + ebase + ci * CH2, CH2)],
                                  e_bufs[b], sem_e).wait()

        start(0, 0)

        def pair_body(cp, _):
            for b in range(2):
                ci = cp * 2 + b

                @pl.when(ci + 1 < NCH)
                def _():
                    start(ci + 1, 1 - b)

                wait(ci, b)
                pck_b = pck_bufs[b]
                e_b = e_bufs[b]

                @plsc.parallel_loop(0, CH2, 16, unroll=8)
                def _edges(off):
                    pk = pck_b[pl.ds(off, 16)]
                    s16 = pk >> 14
                    d16 = pk & 16383
                    ev = e_b[pl.ds(off, 16)]
                    for cp in range(KH):
                        w16 = plsc.load_gather(f_t, [d16 + (cp * N)])
                        flo, fhi = plsc.unpack(
                            plsc.bitcast(w16, jnp.bfloat16),
                            format=plsc.PackFormat.INTERLEAVED)
                        plsc.addupdate_scatter(
                            out_t, [s16 + ((2 * cp) * N)], ev * flo)
                        plsc.addupdate_scatter(
                            out_t, [s16 + ((2 * cp + 1) * N)], ev * fhi)

            return 0

        lax.fori_loop(0, NCH // 2, pair_body, 0)

        if edge_split:
            pltpu.sync_copy(
                out_t, out_ref.at[pl.ds((cid * F_all + c0) * N, K * N)])
        else:
            pltpu.sync_copy(out_t, out_ref.at[pl.ds(c0 * N, K * N)])

    return pl.kernel(
        body,
        out_type=jax.ShapeDtypeStruct((NSC * F_all * N,), jnp.float32),
        mesh=_mesh(), compiler_params=_sc_params(),
        scratch_types=scratch)


def _blockdiag(aw):
    H, F = aw.shape
    A = jnp.zeros((8, H * F), jnp.float32)
    for h in range(H):
        A = A.at[h, h * F:(h + 1) * F].set(aw[h])
    return A


def _bcast_col(v):
    return jnp.broadcast_to(v.reshape(-1, 1), (v.size, 128)).astype(jnp.float32)


def kernel(features, edge_index, W1, b1, a1w1, a1b1, a2w1, a2b1,
           W2, b2, a1w2, a1b2, a2w2, a2b2):
    N, DIN = features.shape
    E = edge_index.shape[1]
    H, F1, _ = W1.shape
    F2 = W2.shape[1]
    HF1, HF2 = H * F1, H * F2

    Ws1 = W1.reshape(HF1, DIN)
    Ws2 = W2.reshape(HF2, HF1)
    bs1 = b1.reshape(HF1)
    bs2 = b2.reshape(HF2)
    A1_1, A2_1 = _blockdiag(a1w1), _blockdiag(a2w1)
    A1_2, A2_2 = _blockdiag(a1w2), _blockdiag(a2w2)
    ab1_1 = _bcast_col(jnp.pad(a1b1, (0, 8 - H)))
    ab2_1 = _bcast_col(jnp.pad(a2b1, (0, 8 - H)))
    ab1_2 = _bcast_col(jnp.pad(a1b2, (0, 8 - H)))
    ab2_2 = _bcast_col(jnp.pad(a2b2, (0, 8 - H)))

    att1 = _make_att(N, E, from_packed=False)
    att2 = _make_att(N, E, from_packed=True)
    agg1 = _make_agg(N, E, HF1, 8000, edge_split=True)
    agg2 = _make_agg(N, E, HF2, 8000)

    fp1, a1v1, a2v1 = _dense1(
        features, Ws1[0::2], Ws1[1::2],
        _bcast_col(bs1[0::2]), _bcast_col(bs1[1::2]),
        A1_1[:, 0::2], A1_1[:, 1::2], ab1_1,
        A2_1[:, 0::2], A2_1[:, 1::2], ab2_1)
    e1, spart1, pck = att1(a1v1.reshape(-1), a2v1.reshape(-1),
                           edge_index.reshape(-1))
    out1 = agg1(fp1.reshape(-1), e1, pck)

    fp2, a1v2, a2v2 = _dense2(
        out1.reshape(2 * HF1, N), spart1.reshape(2 * _NW, N),
        Ws2[0::2], Ws2[1::2],
        _bcast_col(bs2[0::2]), _bcast_col(bs2[1::2]),
        A1_2[:, 0::2], A1_2[:, 1::2], ab1_2,
        A2_2[:, 0::2], A2_2[:, 1::2], ab2_2)
    e2, spart2 = att2(a1v2.reshape(-1), a2v2.reshape(-1), pck)
    out2 = agg2(fp2.reshape(-1), e2, pck)

    return _finish(out2.reshape(HF2, N), spart2.reshape(2 * _NW, N))


# speedup vs baseline: 1.4621x; 1.0179x over previous
import functools

import jax
import jax.numpy as jnp
from jax import lax
from jax.experimental import pallas as pl
from jax.experimental.pallas import tpu as pltpu
from jax.experimental.pallas import tpu_sc as plsc

_NC = 2
_NS = 16
_NW = _NC * _NS

_BN = 2048


def _mesh():
    return plsc.VectorSubcoreMesh(core_axis_name="c", subcore_axis_name="s")


def _sc_params():
    return pltpu.CompilerParams(needs_layout_passes=False)


def _pack_pair(fe, fo):
    ue = lax.bitcast_convert_type(fe.astype(jnp.bfloat16), jnp.uint16)
    uo = lax.bitcast_convert_type(fo.astype(jnp.bfloat16), jnp.uint16)
    w = (uo.astype(jnp.uint32) << 16) | ue.astype(jnp.uint32)
    return lax.bitcast_convert_type(w, jnp.int32)


def _dense1(x, We, Wo, be, bo, A1e, A1o, ab1, A2e, A2o, ab2):
    N, DIN = x.shape
    HFH = We.shape[0]
    grid = (pl.cdiv(N, _BN),)

    def body(x_ref, we_ref, wo_ref, be_ref, bo_ref,
             a1e_ref, a1o_ref, ab1_ref, a2e_ref, a2o_ref, ab2_ref,
             f_out, a1_out, a2_out):
        x = x_ref[...]
        fe = lax.dot_general(we_ref[...], x, (((1,), (1,)), ((), ())),
                             preferred_element_type=jnp.float32) + be_ref[:, 0:1]
        fo = lax.dot_general(wo_ref[...], x, (((1,), (1,)), ((), ())),
                             preferred_element_type=jnp.float32) + bo_ref[:, 0:1]
        f_out[...] = _pack_pair(fe, fo)
        a1_out[...] = (
            lax.dot_general(a1e_ref[...], fe, (((1,), (0,)), ((), ())),
                            preferred_element_type=jnp.float32)
            + lax.dot_general(a1o_ref[...], fo, (((1,), (0,)), ((), ())),
                              preferred_element_type=jnp.float32)
            + ab1_ref[:, 0:1])
        a2_out[...] = (
            lax.dot_general(a2e_ref[...], fe, (((1,), (0,)), ((), ())),
                            preferred_element_type=jnp.float32)
            + lax.dot_general(a2o_ref[...], fo, (((1,), (0,)), ((), ())),
                              preferred_element_type=jnp.float32)
            + ab2_ref[:, 0:1])

    return pl.pallas_call(
        body,
        grid=grid,
        in_specs=[
            pl.BlockSpec((_BN, DIN), lambda i: (i, 0)),
            pl.BlockSpec((HFH, DIN), lambda i: (0, 0)),
            pl.BlockSpec((HFH, DIN), lambda i: (0, 0)),
            pl.BlockSpec((HFH, 128), lambda i: (0, 0)),
            pl.BlockSpec((HFH, 128), lambda i: (0, 0)),
            pl.BlockSpec((8, HFH), lambda i: (0, 0)),
            pl.BlockSpec((8, HFH), lambda i: (0, 0)),
            pl.BlockSpec((8, 128), lambda i: (0, 0)),
            pl.BlockSpec((8, HFH), lambda i: (0, 0)),
            pl.BlockSpec((8, HFH), lambda i: (0, 0)),
            pl.BlockSpec((8, 128), lambda i: (0, 0)),
        ],
        out_specs=[
            pl.BlockSpec((HFH, _BN), lambda i: (0, i)),
            pl.BlockSpec((8, _BN), lambda i: (0, i)),
            pl.BlockSpec((8, _BN), lambda i: (0, i)),
        ],
        out_shape=[
            jax.ShapeDtypeStruct((HFH, N), jnp.int32),
            jax.ShapeDtypeStruct((8, N), jnp.float32),
            jax.ShapeDtypeStruct((8, N), jnp.float32),
        ],
    )(x, We, Wo, be, bo, A1e, A1o, ab1, A2e, A2o, ab2)


def _dense2(xP, spart, We, Wo, be, bo, A1e, A1o, ab1, A2e, A2o, ab2):
    F = xP.shape[0] // 2
    N = xP.shape[1]
    R = spart.shape[0]
    HFH = We.shape[0]
    FH = F // 2
    grid = (pl.cdiv(N, _BN),)

    def body(x0_ref, x1_ref, s_ref, we_ref, wo_ref, be_ref, bo_ref,
             a1e_ref, a1o_ref, ab1_ref, a2e_ref, a2o_ref, ab2_ref,
             f_out, a1_out, a2_out):
        s = s_ref[...]
        inv0 = 1.0 / jnp.maximum(
            jnp.sum(s[:R // 2], axis=0, keepdims=True), 1e-30)
        inv1 = 1.0 / jnp.maximum(
            jnp.sum(s[R // 2:], axis=0, keepdims=True), 1e-30)
        scale = jnp.concatenate(
            [jnp.broadcast_to(inv0, (FH, inv0.shape[1])),
             jnp.broadcast_to(inv1, (FH, inv1.shape[1]))], axis=0)
        x2 = jnp.maximum(x0_ref[...] + x1_ref[...], 0.0) * scale
        fe = lax.dot_general(we_ref[...], x2, (((1,), (0,)), ((), ())),
                             preferred_element_type=jnp.float32) + be_ref[:, 0:1]
        fo = lax.dot_general(wo_ref[...], x2, (((1,), (0,)), ((), ())),
                             preferred_element_type=jnp.float32) + bo_ref[:, 0:1]
        f_out[...] = _pack_pair(fe, fo)
        a1_out[...] = (
            lax.dot_general(a1e_ref[...], fe, (((1,), (0,)), ((), ())),
                            preferred_element_type=jnp.float32)
            + lax.dot_general(a1o_ref[...], fo, (((1,), (0,)), ((), ())),
                              preferred_element_type=jnp.float32)
            + ab1_ref[:, 0:1])
        a2_out[...] = (
            lax.dot_general(a2e_ref[...], fe, (((1,), (0,)), ((), ())),
                            preferred_element_type=jnp.float32)
            + lax.dot_general(a2o_ref[...], fo, (((1,), (0,)), ((), ())),
                              preferred_element_type=jnp.float32)
            + ab2_ref[:, 0:1])

    return pl.pallas_call(
        body,
        grid=grid,
        in_specs=[
            pl.BlockSpec((F, _BN), lambda i: (0, i)),
            pl.BlockSpec((F, _BN), lambda i: (1, i)),
            pl.BlockSpec((R, _BN), lambda i: (0, i)),
            pl.BlockSpec((HFH, F), lambda i: (0, 0)),
            pl.BlockSpec((HFH, F), lambda i: (0, 0)),
            pl.BlockSpec((HFH, 128), lambda i: (0, 0)),
            pl.BlockSpec((HFH, 128), lambda i: (0, 0)),
            pl.BlockSpec((8, HFH), lambda i: (0, 0)),
            pl.BlockSpec((8, HFH), lambda i: (0, 0)),
            pl.BlockSpec((8, 128), lambda i: (0, 0)),
            pl.BlockSpec((8, HFH), lambda i: (0, 0)),
            pl.BlockSpec((8, HFH), lambda i: (0, 0)),
            pl.BlockSpec((8, 128), lambda i: (0, 0)),
        ],
        out_specs=[
            pl.BlockSpec((HFH, _BN), lambda i: (0, i)),
            pl.BlockSpec((8, _BN), lambda i: (0, i)),
            pl.BlockSpec((8, _BN), lambda i: (0, i)),
        ],
        out_shape=[
            jax.ShapeDtypeStruct((HFH, N), jnp.int32),
            jax.ShapeDtypeStruct((8, N), jnp.float32),
            jax.ShapeDtypeStruct((8, N), jnp.float32),
        ],
    )(xP, xP, spart, We, Wo, be, bo, A1e, A1o, ab1, A2e, A2o, ab2)


def _finish(pT, spart):
    F, N = pT.shape
    R = spart.shape[0]
    FH = F // 2
    BNf = 512
    grid = (pl.cdiv(N, BNf),)

    def body(p_ref, s_ref, o_ref):
        s = s_ref[...]
        inv0 = 1.0 / jnp.maximum(
            jnp.sum(s[:R // 2], axis=0, keepdims=True), 1e-30)
        inv1 = 1.0 / jnp.maximum(
            jnp.sum(s[R // 2:], axis=0, keepdims=True), 1e-30)
        scale = jnp.concatenate(
            [jnp.broadcast_to(inv0, (FH, inv0.shape[1])),
             jnp.broadcast_to(inv1, (FH, inv1.shape[1]))], axis=0)
        y = jnp.maximum(p_ref[...], 0.0) * scale
        o_ref[...] = y.T

    return pl.pallas_call(
        body,
        grid=grid,
        in_specs=[pl.BlockSpec((F, BNf), lambda i: (0, i)),
                  pl.BlockSpec((R, BNf), lambda i: (0, i))],
        out_specs=pl.BlockSpec((BNf, F), lambda i: (i, 0)),
        out_shape=jax.ShapeDtypeStruct((N, F), jnp.float32),
    )(pT, spart)


def _make_att(N, E, from_packed):
    CH = E // _NW
    twoN = 2 * N

    out_type = [
        jax.ShapeDtypeStruct((2 * E,), jnp.float32),
        jax.ShapeDtypeStruct((2 * _NW * N,), jnp.float32),
    ]
    scratch = [
        pltpu.VMEM((twoN,), jnp.float32),
        pltpu.VMEM((twoN,), jnp.float32),
        pltpu.VMEM((twoN,), jnp.float32),
        pltpu.VMEM((2 * CH,), jnp.float32),
        pltpu.VMEM((CH,), jnp.int32),
    ]
    if not from_packed:
        out_type.append(jax.ShapeDtypeStruct((E,), jnp.int32))
        scratch.append(pltpu.VMEM((CH,), jnp.int32))
        scratch.append(pltpu.VMEM((CH,), jnp.int32))

    def body(av1_ref, av2_ref, edges_ref, *refs):
        if from_packed:
            e_out, s_out, a1t, a2t, s_loc, e_v, pck_v = refs
        else:
            (e_out, s_out, pck_out,
             a1t, a2t, s_loc, e_v, pck_v, src_v, dst_v) = refs
        wid = lax.axis_index("s") * _NC + lax.axis_index("c")
        base = wid * CH

        pltpu.sync_copy(av1_ref.at[pl.ds(0, twoN)], a1t)
        pltpu.sync_copy(av2_ref.at[pl.ds(0, twoN)], a2t)
        if from_packed:
            pltpu.sync_copy(edges_ref.at[pl.ds(base, CH)], pck_v)
        else:
            pltpu.sync_copy(edges_ref.at[pl.ds(base, CH)], src_v)
            pltpu.sync_copy(edges_ref.at[pl.ds(E + base, CH)], dst_v)

        zero16 = jnp.zeros((16,), jnp.float32)

        @plsc.parallel_loop(0, twoN, 16, unroll=8)
        def _zero(off):
            s_loc[pl.ds(off, 16)] = zero16

        @plsc.parallel_loop(0, CH, 16, unroll=8)
        def _edges(off):
            if from_packed:
                pk = pck_v[pl.ds(off, 16)]
                s16 = pk >> 14
                d16 = pk & 16383
            else:
                s16 = src_v[pl.ds(off, 16)]
                d16 = dst_v[pl.ds(off, 16)]
                pck_v[pl.ds(off, 16)] = (s16 << 14) | d16
            for h in range(2):
                a1v = plsc.load_gather(a1t, [s16 + (h * N)])
                a2v = plsc.load_gather(a2t, [d16 + (h * N)])
                v = a1v + a2v
                v = jnp.where(v > 0.0, v, 0.01 * v)
                ev = jnp.exp(v)
                e_v[pl.ds(h * CH + off, 16)] = ev
                plsc.addupdate_scatter(s_loc, [s16 + (h * N)], ev)

        pltpu.sync_copy(e_v.at[pl.ds(0, CH)], e_out.at[pl.ds(base, CH)])
        pltpu.sync_copy(e_v.at[pl.ds(CH, CH)], e_out.at[pl.ds(E + base, CH)])
        pltpu.sync_copy(s_loc.at[pl.ds(0, N)],
                        s_out.at[pl.ds(wid * N, N)])
        pltpu.sync_copy(s_loc.at[pl.ds(N, N)],
                        s_out.at[pl.ds(_NW * N + wid * N, N)])
        if not from_packed:
            pltpu.sync_copy(pck_v, pck_out.at[pl.ds(base, CH)])

    return pl.kernel(body, out_type=tuple(out_type), mesh=_mesh(),
                     compiler_params=_sc_params(), scratch_types=scratch)


def _make_agg(N, E, F_all, CH2, edge_split=False):
    NSC = 2 if edge_split else 1
    TILES = _NS if edge_split else _NW
    K = F_all // TILES
    KH = K // 2
    half = F_all // 2
    ESC = E // NSC

    NCH = ESC // CH2

    scratch = [
        pltpu.VMEM((KH * N,), jnp.int32),
        pltpu.VMEM((K * N,), jnp.float32),
        pltpu.VMEM((CH2,), jnp.int32),
        pltpu.VMEM((CH2,), jnp.int32),
        pltpu.VMEM((CH2,), jnp.float32),
        pltpu.VMEM((CH2,), jnp.float32),
        pltpu.SemaphoreType.DMA,
        pltpu.SemaphoreType.DMA,
    ]

    def body(fT_ref, e_ref, pck_ref, out_ref,
             f_t, out_t, pck_b0, pck_b1, e_b0, e_b1, sem_p, sem_e):
        sid = lax.axis_index("s")
        cid = lax.axis_index("c")
        if edge_split:
            tid = sid
            ebase = cid * ESC
        else:
            tid = sid * _NC + cid
            ebase = 0
        c0 = tid * K
        h = c0 // half
        pck_bufs = (pck_b0, pck_b1)
        e_bufs = (e_b0, e_b1)

        pltpu.sync_copy(fT_ref.at[pl.ds(tid * (KH * N), KH * N)], f_t)

        zero16 = jnp.zeros((16,), jnp.float32)

        @plsc.parallel_loop(0, K * N, 16, unroll=8)
        def _zero(off):
            out_t[pl.ds(off, 16)] = zero16

        def start(ci, b):
            pltpu.async_copy(pck_ref.at[pl.ds(ebase + ci * CH2, CH2)],
                             pck_bufs[b], sem_p)
            pltpu.async_copy(e_ref.at[pl.ds(h * E + ebase + ci * CH2, CH2)],
                             e_bufs[b], sem_e)

        def wait(ci, b):
            pltpu.make_async_copy(pck_ref.at[pl.ds(ebase + ci * CH2, CH2)],
                                  pck_bufs[b], sem_p).wait()
            pltpu.make_async_copy(e_ref.at[pl.ds(h * E + ebase + ci * CH2, CH2)],
                                  e_bufs[b], sem_e).wait()

        start(0, 0)

        def pair_body(cp, _):
            for b in range(2):
                ci = cp * 2 + b

                @pl.when(ci + 1 < NCH)
                def _():
                    start(ci + 1, 1 - b)

                wait(ci, b)
                pck_b = pck_bufs[b]
                e_b = e_bufs[b]

                @plsc.parallel_loop(0, CH2, 16, unroll=4)
                def _edges(off):
                    pk = pck_b[pl.ds(off, 16)]
                    s16 = pk >> 14
                    d16 = pk & 16383
                    ev = e_b[pl.ds(off, 16)]
                    for cp in range(KH):
                        w16 = plsc.load_gather(f_t, [d16 + (cp * N)])
                        flo, fhi = plsc.unpack(
                            plsc.bitcast(w16, jnp.bfloat16),
                            format=plsc.PackFormat.INTERLEAVED)
                        plsc.addupdate_scatter(
                            out_t, [s16 + ((2 * cp) * N)], ev * flo)
                        plsc.addupdate_scatter(
                            out_t, [s16 + ((2 * cp + 1) * N)], ev * fhi)

            return 0

        lax.fori_loop(0, NCH // 2, pair_body, 0)

        if edge_split:
            pltpu.sync_copy(
                out_t, out_ref.at[pl.ds((cid * F_all + c0) * N, K * N)])
        else:
            pltpu.sync_copy(out_t, out_ref.at[pl.ds(c0 * N, K * N)])

    return pl.kernel(
        body,
        out_type=jax.ShapeDtypeStruct((NSC * F_all * N,), jnp.float32),
        mesh=_mesh(), compiler_params=_sc_params(),
        scratch_types=scratch)


def _blockdiag(aw):
    H, F = aw.shape
    A = jnp.zeros((8, H * F), jnp.float32)
    for h in range(H):
        A = A.at[h, h * F:(h + 1) * F].set(aw[h])
    return A


def _bcast_col(v):
    return jnp.broadcast_to(v.reshape(-1, 1), (v.size, 128)).astype(jnp.float32)


def kernel(features, edge_index, W1, b1, a1w1, a1b1, a2w1, a2b1,
           W2, b2, a1w2, a1b2, a2w2, a2b2):
    N, DIN = features.shape
    E = edge_index.shape[1]
    H, F1, _ = W1.shape
    F2 = W2.shape[1]
    HF1, HF2 = H * F1, H * F2

    Ws1 = W1.reshape(HF1, DIN)
    Ws2 = W2.reshape(HF2, HF1)
    bs1 = b1.reshape(HF1)
    bs2 = b2.reshape(HF2)
    A1_1, A2_1 = _blockdiag(a1w1), _blockdiag(a2w1)
    A1_2, A2_2 = _blockdiag(a1w2), _blockdiag(a2w2)
    ab1_1 = _bcast_col(jnp.pad(a1b1, (0, 8 - H)))
    ab2_1 = _bcast_col(jnp.pad(a2b1, (0, 8 - H)))
    ab1_2 = _bcast_col(jnp.pad(a1b2, (0, 8 - H)))
    ab2_2 = _bcast_col(jnp.pad(a2b2, (0, 8 - H)))

    att1 = _make_att(N, E, from_packed=False)
    att2 = _make_att(N, E, from_packed=True)
    agg1 = _make_agg(N, E, HF1, 8000, edge_split=True)
    agg2 = _make_agg(N, E, HF2, 8000)

    fp1, a1v1, a2v1 = _dense1(
        features, Ws1[0::2], Ws1[1::2],
        _bcast_col(bs1[0::2]), _bcast_col(bs1[1::2]),
        A1_1[:, 0::2], A1_1[:, 1::2], ab1_1,
        A2_1[:, 0::2], A2_1[:, 1::2], ab2_1)
    e1, spart1, pck = att1(a1v1.reshape(-1), a2v1.reshape(-1),
                           edge_index.reshape(-1))
    out1 = agg1(fp1.reshape(-1), e1, pck)

    fp2, a1v2, a2v2 = _dense2(
        out1.reshape(2 * HF1, N), spart1.reshape(2 * _NW, N),
        Ws2[0::2], Ws2[1::2],
        _bcast_col(bs2[0::2]), _bcast_col(bs2[1::2]),
        A1_2[:, 0::2], A1_2[:, 1::2], ab1_2,
        A2_2[:, 0::2], A2_2[:, 1::2], ab2_2)
    e2, spart2 = att2(a1v2.reshape(-1), a2v2.reshape(-1), pck)
    out2 = agg2(fp2.reshape(-1), e2, pck)

    return _finish(out2.reshape(HF2, N), spart2.reshape(2 * _NW, N))
